# baseline (speedup 1.0000x reference)
import functools

import jax
import jax.numpy as jnp
from jax import lax
from jax.experimental import pallas as pl
from jax.experimental.pallas import tpu as pltpu
from jax.experimental.pallas import tpu_sc as plsc

N_TOK = 16384
N_EXP = 64
D = 768
BM = 64
MP = N_TOK + N_EXP * BM
NT = MP // BM
TRASH = N_TOK
OUT_ROWS = N_TOK + 8
TB = 1024
NSUB = 16
TPW = N_TOK // NSUB
CPW = MP // NSUB
GPW = TPW // 16


def _router_body(x_ref, wr_ref, br_ref, routes_ref, rpm_ref, rps_ref, cnt_ref,
                 xbf_ref):
    i = pl.program_id(0)
    x = x_ref[...]
    xbf_ref[...] = x.astype(jnp.bfloat16)
    wr = wr_ref[...]
    logits = lax.dot_general(x, wr, (((1,), (1,)), ((), ())),
                             preferred_element_type=jnp.float32)
    logits = logits + br_ref[...]
    prob = jax.nn.softmax(logits, axis=-1)
    rpm = jnp.max(prob, axis=-1)
    eiota = lax.broadcasted_iota(jnp.int32, (TB, N_EXP), 1)
    routes = jnp.min(jnp.where(prob == rpm[:, None], eiota, N_EXP), axis=-1)
    onehot = (eiota == routes[:, None]).astype(jnp.float32)
    routes_ref[...] = routes.reshape(TB // 128, 128)
    rpm_ref[...] = rpm.reshape(TB // 128, 128)

    @pl.when(i == 0)
    def _():
        rps_ref[...] = jnp.zeros_like(rps_ref)
        cnt_ref[...] = jnp.zeros_like(cnt_ref)

    rps_ref[...] += jnp.sum(prob, axis=0).reshape(1, N_EXP)
    cnt_ref[...] += jnp.sum(onehot, axis=0).reshape(1, N_EXP)


def _router(x, Wr, br):
    n_blk = N_TOK // TB
    return pl.pallas_call(
        _router_body,
        grid=(n_blk,),
        in_specs=[
            pl.BlockSpec((TB, D), lambda i: (i, 0)),
            pl.BlockSpec((N_EXP, D), lambda i: (0, 0)),
            pl.BlockSpec((1, N_EXP), lambda i: (0, 0)),
        ],
        out_specs=[
            pl.BlockSpec((TB // 128, 128), lambda i: (i, 0)),
            pl.BlockSpec((TB // 128, 128), lambda i: (i, 0)),
            pl.BlockSpec((1, N_EXP), lambda i: (0, 0)),
            pl.BlockSpec((1, N_EXP), lambda i: (0, 0)),
            pl.BlockSpec((TB, D), lambda i: (i, 0)),
        ],
        out_shape=[
            jax.ShapeDtypeStruct((N_TOK // 128, 128), jnp.int32),
            jax.ShapeDtypeStruct((N_TOK // 128, 128), jnp.float32),
            jax.ShapeDtypeStruct((1, N_EXP), jnp.float32),
            jax.ShapeDtypeStruct((1, N_EXP), jnp.float32),
            jax.ShapeDtypeStruct((N_TOK, D), jnp.bfloat16),
        ],
    )(x, Wr, br.reshape(1, N_EXP))


def _bin_kernel_body(routes_hbm, rpm2_hbm, base_hbm,
                     gid_hbm, dest_hbm, scal_hbm,
                     routes_v, hist_v, cnt_v, allh_v,
                     base_v, zero_v, trash_v, slots_v, gvals_v, rpm2_v,
                     hist_s, cnt_s, hist_sh, sem):
    s = lax.axis_index("s")
    tok0 = s * TPW
    cb = s * CPW
    iot = lax.iota(jnp.int32, 16)

    @pl.loop(0, CPW // 16)
    def _(k):
        zero_v[pl.ds(16 * k, 16)] = jnp.zeros((16,), jnp.int32)
        trash_v[pl.ds(16 * k, 16)] = jnp.full((16,), TRASH, jnp.int32)

    init0 = pltpu.async_copy(zero_v, gid_hbm.at[pl.ds(cb, CPW)], sem)
    init1 = pltpu.async_copy(trash_v, dest_hbm.at[pl.ds(cb, CPW)], sem)

    pltpu.sync_copy(routes_hbm.at[pl.ds(tok0, TPW)], routes_v)
    pltpu.sync_copy(rpm2_hbm.at[s], rpm2_v)
    pltpu.sync_copy(base_hbm, base_v)

    for l in range(N_EXP):
        hist_s[l] = 0

    @pl.loop(0, TPW // 16)
    def _(g):
        ev = routes_v[pl.ds(g * 16, 16)]
        for l in range(16):
            e = ev[l]
            hist_s[e] = hist_s[e] + 1

    for j in range(4):
        acc = jnp.zeros((16,), jnp.int32)
        for l in range(16):
            hs = hist_s[j * 16 + l]
            acc = jnp.where(iot == l, lax.broadcast(hs, (16,)), acc)
        hist_v[pl.ds(j * 16, 16)] = acc

    pltpu.sync_copy(hist_v, hist_sh.at[pl.ds(s * N_EXP, N_EXP)])
    init0.wait()
    init1.wait()
    plsc.subcore_barrier()
    pltpu.sync_copy(hist_sh, allh_v)

    for j in range(4):
        pre = jnp.zeros((16,), jnp.int32)
        for sp in range(NSUB - 1):
            h = allh_v[pl.ds(sp * N_EXP + j * 16, 16)]
            m = lax.broadcast((s > sp).astype(jnp.int32), (16,))
            pre = pre + h * m
        cv = base_v[pl.ds(j * 16, 16)] + pre
        cnt_v[pl.ds(j * 16, 16)] = cv
        for l in range(16):
            cnt_s[j * 16 + l] = cv[l]

    for r in range(8):
        @pl.loop(0, 8)
        def _(g2, r=r):
            g = r * 8 + g2
            ev = routes_v[pl.ds(g * 16, 16)]
            slot_acc = jnp.zeros((16,), jnp.int32)
            for l in range(16):
                e = ev[l]
                sl = cnt_s[e]
                cnt_s[e] = sl + 1
                slot_acc = jnp.where(iot == l, lax.broadcast(sl, (16,)),
                                     slot_acc)
            c0 = g2 * 16
            slots_v[r, pl.ds(c0, 16)] = slot_acc
            gvals_v[r, pl.ds(c0, 16)] = iot + (tok0 + g * 16)

    for r in range(8):
        pltpu.async_copy(gvals_v.at[r], gid_hbm.at[slots_v.at[r]], sem).wait()
        pltpu.async_copy(gvals_v.at[r], dest_hbm.at[slots_v.at[r]], sem).wait()
        pltpu.async_copy(rpm2_v.at[r], scal_hbm.at[slots_v.at[r]], sem).wait()


def _binning(routes, rpm, base):
    mesh = plsc.VectorSubcoreMesh(core_axis_name="c", subcore_axis_name="s",
                                  num_cores=1)
    f = pl.kernel(
        _bin_kernel_body,
        out_type=[
            jax.ShapeDtypeStruct((MP,), jnp.int32),
            jax.ShapeDtypeStruct((MP,), jnp.int32),
            jax.ShapeDtypeStruct((MP,), jnp.float32),
        ],
        mesh=mesh,
        scratch_types=[
            pltpu.VMEM((TPW,), jnp.int32),
            pltpu.VMEM((N_EXP,), jnp.int32),
            pltpu.VMEM((N_EXP,), jnp.int32),
            pltpu.VMEM((NSUB * N_EXP,), jnp.int32),
            pltpu.VMEM((N_EXP,), jnp.int32),
            pltpu.VMEM((CPW,), jnp.int32),
            pltpu.VMEM((CPW,), jnp.int32),
            pltpu.VMEM((8, 128), jnp.int32),
            pltpu.VMEM((8, 128), jnp.int32),
            pltpu.VMEM((8, 128), jnp.float32),
            pltpu.SMEM((N_EXP,), jnp.int32),
            pltpu.SMEM((N_EXP,), jnp.int32),
            pltpu.VMEM_SHARED((NSUB * N_EXP,), jnp.int32),
            pltpu.SemaphoreType.DMA,
        ],
    )
    return f(routes, rpm.reshape(NSUB, 8, 128), base)


_SPW = MP // 32
_CH = 64
_NCH = _SPW // _CH


def _gather_body(x_hbm, gid2_hbm, xs_hbm, gid2_v, rows0, rows1,
                 g0, g1, w0, w1):
    w = lax.axis_index("s") * 2 + lax.axis_index("c")
    pltpu.sync_copy(gid2_hbm.at[w], gid2_v)
    rows = (rows0, rows1)
    gsem = (g0, g1)
    wsem = (w0, w1)

    @pl.loop(0, _NCH, step=2)
    def _(j):
        gds = []
        for b in range(2):
            c = j + b

            @pl.when(j > 0)
            def _(b=b, c=c):
                pltpu.make_async_copy(
                    rows[b], xs_hbm.at[pl.ds(w * _SPW + c * _CH, _CH)],
                    wsem[b]).wait()

            gds.append(pltpu.async_copy(
                x_hbm.at[gid2_v.at[c]], rows[b], gsem[b]))
        for b in range(2):
            c = j + b
            gds[b].wait()
            pltpu.async_copy(
                rows[b], xs_hbm.at[pl.ds(w * _SPW + c * _CH, _CH)], wsem[b])

    for b in range(2):
        pltpu.make_async_copy(
            rows[b], xs_hbm.at[pl.ds(w * _SPW, _CH)], wsem[b]).wait()


_D2 = D // 2


def _bf16_to_words(a):
    n = a.shape[0]
    return lax.bitcast_convert_type(a.reshape(n, _D2, 2), jnp.int32)


def _words_to_bf16(a):
    return lax.bitcast_convert_type(a, jnp.bfloat16).reshape(a.shape[0], D)


def _gather(xw, gid):
    mesh = plsc.VectorSubcoreMesh(core_axis_name="c", subcore_axis_name="s")
    f = pl.kernel(
        _gather_body,
        out_type=jax.ShapeDtypeStruct((MP, _D2), jnp.int32),
        mesh=mesh,
        scratch_types=[
            pltpu.VMEM((_NCH, _CH), jnp.int32),
            pltpu.VMEM((_CH, _D2), jnp.int32),
            pltpu.VMEM((_CH, _D2), jnp.int32),
            pltpu.SemaphoreType.DMA,
            pltpu.SemaphoreType.DMA,
            pltpu.SemaphoreType.DMA,
            pltpu.SemaphoreType.DMA,
        ],
    )
    return f(xw, gid.reshape(32, _NCH, _CH))


def _ffn_body(te_ref, xs_ref, we_ref, be_ref, sc_ref, ys_ref):
    w = we_ref[0]
    y = lax.dot_general(xs_ref[...], w, (((1,), (1,)), ((), ())),
                        preferred_element_type=jnp.float32)
    y = jnp.maximum(y + be_ref[0], 0.0) * sc_ref[...]
    ys_ref[...] = y.astype(jnp.bfloat16)


def _ffn(xs, We, be, scal, te):
    grid_spec = pltpu.PrefetchScalarGridSpec(
        num_scalar_prefetch=1,
        grid=(NT,),
        in_specs=[
            pl.BlockSpec((BM, D), lambda i, te: (i, 0)),
            pl.BlockSpec((1, D, D), lambda i, te: (te[i], 0, 0)),
            pl.BlockSpec((1, 1, D), lambda i, te: (te[i], 0, 0)),
            pl.BlockSpec((BM, 1), lambda i, te: (i, 0)),
        ],
        out_specs=pl.BlockSpec((BM, D), lambda i, te: (i, 0)),
    )
    return pl.pallas_call(
        _ffn_body,
        grid_spec=grid_spec,
        out_shape=jax.ShapeDtypeStruct((MP, D), jnp.bfloat16),
    )(te, xs, We.astype(jnp.bfloat16), be.reshape(N_EXP, 1, D),
      scal.reshape(MP, 1))


def _scatter_body(ys_hbm, did2_hbm, out_hbm, did2_v, rows0, rows1,
                  g0, g1, w0, w1):
    w = lax.axis_index("s") * 2 + lax.axis_index("c")
    pltpu.sync_copy(did2_hbm.at[w], did2_v)
    rows = (rows0, rows1)
    rsem = (g0, g1)
    wsem = (w0, w1)

    @pl.loop(0, _NCH, step=2)
    def _(j):
        rds = []
        for b in range(2):
            c = j + b

            @pl.when(j > 0)
            def _(b=b, c=c):
                pltpu.make_async_copy(
                    rows[b], out_hbm.at[did2_v.at[c]], wsem[b]).wait()

            rds.append(pltpu.async_copy(
                ys_hbm.at[pl.ds(w * _SPW + c * _CH, _CH)], rows[b], rsem[b]))
        for b in range(2):
            c = j + b
            rds[b].wait()
            pltpu.async_copy(rows[b], out_hbm.at[did2_v.at[c]], wsem[b])

    for b in range(2):
        pltpu.make_async_copy(rows[b], out_hbm.at[did2_v.at[0]],
                              wsem[b]).wait()


def _scatter(ys, dest):
    mesh = plsc.VectorSubcoreMesh(core_axis_name="c", subcore_axis_name="s")
    f = pl.kernel(
        _scatter_body,
        out_type=jax.ShapeDtypeStruct((OUT_ROWS, _D2), jnp.int32),
        mesh=mesh,
        scratch_types=[
            pltpu.VMEM((_NCH, _CH), jnp.int32),
            pltpu.VMEM((_CH, _D2), jnp.int32),
            pltpu.VMEM((_CH, _D2), jnp.int32),
            pltpu.SemaphoreType.DMA,
            pltpu.SemaphoreType.DMA,
            pltpu.SemaphoreType.DMA,
            pltpu.SemaphoreType.DMA,
        ],
    )
    return f(ys, dest.reshape(32, _NCH, _CH))


def kernel(x, Wr, br, We, be):
    routes2, rpm2, rps2, cnt2, xbf = _router(x, Wr, br)
    routes = routes2.reshape(N_TOK)
    rpm = rpm2.reshape(N_TOK)
    counts = cnt2.reshape(N_EXP)

    ci = counts.astype(jnp.int32)
    padded = (ci + (BM - 1)) // BM * BM
    ends = jnp.cumsum(padded)
    base = (ends - padded).astype(jnp.int32)
    te = jnp.searchsorted(ends, jnp.arange(NT, dtype=jnp.int32) * BM,
                          side="right").astype(jnp.int32)
    te = jnp.minimum(te, N_EXP - 1)

    gid, dest, scal = _binning(routes, rpm, base)
    xsw = _gather(_bf16_to_words(xbf), gid)
    ys = _ffn(_words_to_bf16(xsw), We, be, scal, te)
    outb = _scatter(_bf16_to_words(ys), dest)
    final = _words_to_bf16(outb)[:N_TOK].astype(jnp.float32)
    return final, counts, rps2.reshape(N_EXP), rpm


# speedup vs baseline: 2.3797x; 2.3797x over previous
import functools

import jax
import jax.numpy as jnp
from jax import lax
from jax.experimental import pallas as pl
from jax.experimental.pallas import tpu as pltpu
from jax.experimental.pallas import tpu_sc as plsc

N_TOK = 16384
N_EXP = 64
D = 768
BM = 64
MP = N_TOK + N_EXP * BM
NT = MP // BM
TRASH = N_TOK
OUT_ROWS = N_TOK + 8
TB = 1024
NSUB = 16
TPW = N_TOK // NSUB
CPW = MP // NSUB
GPW = TPW // 16


_D2 = D // 2
_MASK_HI = -65536


def _pack_words(a):
    lo = lax.bitcast_convert_type(
        a[:, :_D2].astype(jnp.bfloat16).astype(jnp.float32), jnp.int32)
    hi = lax.bitcast_convert_type(
        a[:, _D2:].astype(jnp.bfloat16).astype(jnp.float32), jnp.int32)
    return lax.shift_right_logical(lo, 16) | (hi & _MASK_HI)


def _unpack_words_f32(w):
    lo = lax.bitcast_convert_type(lax.shift_left(w, 16), jnp.float32)
    hi = lax.bitcast_convert_type(w & _MASK_HI, jnp.float32)
    return lo, hi


def _router_body(x_ref, wr_ref, br_ref, routes_ref, rpm_ref, rps_ref, cnt_ref,
                 xw_ref):
    i = pl.program_id(0)
    x = x_ref[...]
    xw_ref[...] = _pack_words(x)
    wr = wr_ref[...]
    logits = lax.dot_general(x, wr, (((1,), (1,)), ((), ())),
                             preferred_element_type=jnp.float32)
    logits = logits + br_ref[...]
    prob = jax.nn.softmax(logits, axis=-1)
    rpm = jnp.max(prob, axis=-1)
    eiota = lax.broadcasted_iota(jnp.int32, (TB, N_EXP), 1)
    routes = jnp.min(jnp.where(prob == rpm[:, None], eiota, N_EXP), axis=-1)
    onehot = (eiota == routes[:, None]).astype(jnp.float32)
    routes_ref[...] = routes.reshape(TB // 128, 128)
    rpm_ref[...] = rpm.reshape(TB // 128, 128)

    @pl.when(i == 0)
    def _():
        rps_ref[...] = jnp.zeros_like(rps_ref)
        cnt_ref[...] = jnp.zeros_like(cnt_ref)

    rps_ref[...] += jnp.sum(prob, axis=0).reshape(1, N_EXP)
    cnt_ref[...] += jnp.sum(onehot, axis=0).reshape(1, N_EXP)


def _router(x, Wr, br):
    n_blk = N_TOK // TB
    return pl.pallas_call(
        _router_body,
        grid=(n_blk,),
        in_specs=[
            pl.BlockSpec((TB, D), lambda i: (i, 0)),
            pl.BlockSpec((N_EXP, D), lambda i: (0, 0)),
            pl.BlockSpec((1, N_EXP), lambda i: (0, 0)),
        ],
        out_specs=[
            pl.BlockSpec((TB // 128, 128), lambda i: (i, 0)),
            pl.BlockSpec((TB // 128, 128), lambda i: (i, 0)),
            pl.BlockSpec((1, N_EXP), lambda i: (0, 0)),
            pl.BlockSpec((1, N_EXP), lambda i: (0, 0)),
            pl.BlockSpec((TB, _D2), lambda i: (i, 0)),
        ],
        out_shape=[
            jax.ShapeDtypeStruct((N_TOK // 128, 128), jnp.int32),
            jax.ShapeDtypeStruct((N_TOK // 128, 128), jnp.float32),
            jax.ShapeDtypeStruct((1, N_EXP), jnp.float32),
            jax.ShapeDtypeStruct((1, N_EXP), jnp.float32),
            jax.ShapeDtypeStruct((N_TOK, _D2), jnp.int32),
        ],
    )(x, Wr, br.reshape(1, N_EXP))


def _bin_kernel_body(routes_hbm, base_hbm,
                     gid_hbm, dest_hbm,
                     routes_v, hist_v, cnt_v, allh_v,
                     base_v, zero_v, trash_v, slots_v, gvals_v,
                     hist_s, cnt_s, hist_sh, sem):
    s = lax.axis_index("s")
    tok0 = s * TPW
    cb = s * CPW
    iot = lax.iota(jnp.int32, 16)

    @pl.loop(0, CPW // 16)
    def _(k):
        zero_v[pl.ds(16 * k, 16)] = jnp.zeros((16,), jnp.int32)
        trash_v[pl.ds(16 * k, 16)] = jnp.full((16,), TRASH, jnp.int32)

    init0 = pltpu.async_copy(zero_v, gid_hbm.at[pl.ds(cb, CPW)], sem)
    init1 = pltpu.async_copy(trash_v, dest_hbm.at[pl.ds(cb, CPW)], sem)

    pltpu.sync_copy(routes_hbm.at[pl.ds(tok0, TPW)], routes_v)
    pltpu.sync_copy(base_hbm, base_v)

    for l in range(N_EXP):
        hist_s[l] = 0

    @pl.loop(0, TPW // 16)
    def _(g):
        ev = routes_v[pl.ds(g * 16, 16)]
        for l in range(16):
            e = ev[l]
            hist_s[e] = hist_s[e] + 1

    for j in range(4):
        acc = jnp.zeros((16,), jnp.int32)
        for l in range(16):
            hs = hist_s[j * 16 + l]
            acc = jnp.where(iot == l, lax.broadcast(hs, (16,)), acc)
        hist_v[pl.ds(j * 16, 16)] = acc

    pltpu.sync_copy(hist_v, hist_sh.at[pl.ds(s * N_EXP, N_EXP)])
    init0.wait()
    init1.wait()
    plsc.subcore_barrier()
    pltpu.sync_copy(hist_sh, allh_v)

    for j in range(4):
        pre = jnp.zeros((16,), jnp.int32)
        for sp in range(NSUB - 1):
            h = allh_v[pl.ds(sp * N_EXP + j * 16, 16)]
            m = lax.broadcast((s > sp).astype(jnp.int32), (16,))
            pre = pre + h * m
        cv = base_v[pl.ds(j * 16, 16)] + pre
        cnt_v[pl.ds(j * 16, 16)] = cv
        for l in range(16):
            cnt_s[j * 16 + l] = cv[l]

    for r in range(8):
        @pl.loop(0, 8)
        def _(g2, r=r):
            g = r * 8 + g2
            ev = routes_v[pl.ds(g * 16, 16)]
            slot_acc = jnp.zeros((16,), jnp.int32)
            for l in range(16):
                e = ev[l]
                sl = cnt_s[e]
                cnt_s[e] = sl + 1
                slot_acc = jnp.where(iot == l, lax.broadcast(sl, (16,)),
                                     slot_acc)
            c0 = g2 * 16
            slots_v[r, pl.ds(c0, 16)] = slot_acc
            gvals_v[r, pl.ds(c0, 16)] = iot + (tok0 + g * 16)

    for r in range(8):
        pltpu.async_copy(gvals_v.at[r], gid_hbm.at[slots_v.at[r]], sem).wait()
        pltpu.async_copy(gvals_v.at[r], dest_hbm.at[slots_v.at[r]], sem).wait()


def _binning(routes, base):
    mesh = plsc.VectorSubcoreMesh(core_axis_name="c", subcore_axis_name="s",
                                  num_cores=1)
    f = pl.kernel(
        _bin_kernel_body,
        out_type=[
            jax.ShapeDtypeStruct((MP,), jnp.int32),
            jax.ShapeDtypeStruct((MP,), jnp.int32),
        ],
        mesh=mesh,
        scratch_types=[
            pltpu.VMEM((TPW,), jnp.int32),
            pltpu.VMEM((N_EXP,), jnp.int32),
            pltpu.VMEM((N_EXP,), jnp.int32),
            pltpu.VMEM((NSUB * N_EXP,), jnp.int32),
            pltpu.VMEM((N_EXP,), jnp.int32),
            pltpu.VMEM((CPW,), jnp.int32),
            pltpu.VMEM((CPW,), jnp.int32),
            pltpu.VMEM((8, 128), jnp.int32),
            pltpu.VMEM((8, 128), jnp.int32),
            pltpu.SMEM((N_EXP,), jnp.int32),
            pltpu.SMEM((N_EXP,), jnp.int32),
            pltpu.VMEM_SHARED((NSUB * N_EXP,), jnp.int32),
            pltpu.SemaphoreType.DMA,
        ],
    )
    return f(routes, base)


_SPW = MP // 32
_CH = 32
_NCH = _SPW // _CH
_NBUF = 4


def _gather_body(x_hbm, gid2_hbm, xs_hbm, gid2_v, *bufsem):
    w = lax.axis_index("s") * 2 + lax.axis_index("c")
    pltpu.sync_copy(gid2_hbm.at[w], gid2_v)
    rows = bufsem[:_NBUF]
    gsem = bufsem[_NBUF:2 * _NBUF]
    wsem = bufsem[2 * _NBUF:3 * _NBUF]

    @pl.loop(0, _NCH, step=_NBUF)
    def _(j):
        gds = []
        for b in range(_NBUF):
            c = j + b

            @pl.when(j > 0)
            def _(b=b, c=c):
                pltpu.make_async_copy(
                    rows[b], xs_hbm.at[pl.ds(w * _SPW + c * _CH, _CH)],
                    wsem[b]).wait()

            gds.append(pltpu.async_copy(
                x_hbm.at[gid2_v.at[c]], rows[b], gsem[b]))
        for b in range(_NBUF):
            c = j + b
            gds[b].wait()
            pltpu.async_copy(
                rows[b], xs_hbm.at[pl.ds(w * _SPW + c * _CH, _CH)], wsem[b])

    for b in range(_NBUF):
        pltpu.make_async_copy(
            rows[b], xs_hbm.at[pl.ds(w * _SPW, _CH)], wsem[b]).wait()


def _gather(xw, gid):
    mesh = plsc.VectorSubcoreMesh(core_axis_name="c", subcore_axis_name="s")
    f = pl.kernel(
        _gather_body,
        out_type=jax.ShapeDtypeStruct((MP, _D2), jnp.int32),
        mesh=mesh,
        scratch_types=[
            pltpu.VMEM((_NCH, _CH), jnp.int32),
            *[pltpu.VMEM((_CH, _D2), jnp.int32) for _ in range(_NBUF)],
            *[pltpu.SemaphoreType.DMA for _ in range(2 * _NBUF)],
        ],
    )
    return f(xw, gid.reshape(32, _NCH, _CH))


def _ffn_body(te_ref, xs_ref, we_ref, be_ref, ys_ref):
    ww = xs_ref[...]
    lo, hi = _unpack_words_f32(ww)
    w = we_ref[0].astype(jnp.bfloat16)
    y = lax.dot_general(lo.astype(jnp.bfloat16), w[:, :_D2],
                        (((1,), (1,)), ((), ())),
                        preferred_element_type=jnp.float32)
    y += lax.dot_general(hi.astype(jnp.bfloat16), w[:, _D2:],
                         (((1,), (1,)), ((), ())),
                         preferred_element_type=jnp.float32)
    y = jnp.maximum(y + be_ref[0], 0.0)
    ys_ref[...] = _pack_words(y)


def _ffn(xsw, We, be, te):
    grid_spec = pltpu.PrefetchScalarGridSpec(
        num_scalar_prefetch=1,
        grid=(NT,),
        in_specs=[
            pl.BlockSpec((BM, _D2), lambda i, te: (i, 0)),
            pl.BlockSpec((1, D, D), lambda i, te: (te[i], 0, 0)),
            pl.BlockSpec((1, 1, D), lambda i, te: (te[i], 0, 0)),
        ],
        out_specs=pl.BlockSpec((BM, _D2), lambda i, te: (i, 0)),
    )
    return pl.pallas_call(
        _ffn_body,
        grid_spec=grid_spec,
        out_shape=jax.ShapeDtypeStruct((MP, _D2), jnp.int32),
    )(te, xsw, We, be.reshape(N_EXP, 1, D))


def _scatter_body(ys_hbm, did2_hbm, out_hbm, did2_v, *bufsem):
    w = lax.axis_index("s") * 2 + lax.axis_index("c")
    pltpu.sync_copy(did2_hbm.at[w], did2_v)
    rows = bufsem[:_NBUF]
    rsem = bufsem[_NBUF:2 * _NBUF]
    wsem = bufsem[2 * _NBUF:3 * _NBUF]

    @pl.loop(0, _NCH, step=_NBUF)
    def _(j):
        rds = []
        for b in range(_NBUF):
            c = j + b

            @pl.when(j > 0)
            def _(b=b, c=c):
                pltpu.make_async_copy(
                    rows[b], out_hbm.at[did2_v.at[c]], wsem[b]).wait()

            rds.append(pltpu.async_copy(
                ys_hbm.at[pl.ds(w * _SPW + c * _CH, _CH)], rows[b], rsem[b]))
        for b in range(_NBUF):
            c = j + b
            rds[b].wait()
            pltpu.async_copy(rows[b], out_hbm.at[did2_v.at[c]], wsem[b])

    for b in range(_NBUF):
        pltpu.make_async_copy(rows[b], out_hbm.at[did2_v.at[0]],
                              wsem[b]).wait()


def _scatter(ysw, dest):
    mesh = plsc.VectorSubcoreMesh(core_axis_name="c", subcore_axis_name="s")
    f = pl.kernel(
        _scatter_body,
        out_type=jax.ShapeDtypeStruct((OUT_ROWS, _D2), jnp.int32),
        mesh=mesh,
        scratch_types=[
            pltpu.VMEM((_NCH, _CH), jnp.int32),
            *[pltpu.VMEM((_CH, _D2), jnp.int32) for _ in range(_NBUF)],
            *[pltpu.SemaphoreType.DMA for _ in range(2 * _NBUF)],
        ],
    )
    return f(ysw, dest.reshape(32, _NCH, _CH))


def _finish_body(ow_ref, rpm_ref, out_ref):
    lo, hi = _unpack_words_f32(ow_ref[...])
    scale = rpm_ref[...]
    out_ref[:, :_D2] = lo * scale
    out_ref[:, _D2:] = hi * scale


def _finish(outw, rpm_col):
    n_blk = N_TOK // TB
    return pl.pallas_call(
        _finish_body,
        grid=(n_blk,),
        in_specs=[
            pl.BlockSpec((TB, _D2), lambda i: (i, 0)),
            pl.BlockSpec((TB, 1), lambda i: (i, 0)),
        ],
        out_specs=pl.BlockSpec((TB, D), lambda i: (i, 0)),
        out_shape=jax.ShapeDtypeStruct((N_TOK, D), jnp.float32),
    )(outw, rpm_col)


def kernel(x, Wr, br, We, be):
    routes2, rpm2, rps2, cnt2, xw = _router(x, Wr, br)
    routes = routes2.reshape(N_TOK)
    rpm = rpm2.reshape(N_TOK)
    counts = cnt2.reshape(N_EXP)

    ci = counts.astype(jnp.int32)
    padded = (ci + (BM - 1)) // BM * BM
    ends = jnp.cumsum(padded)
    base = (ends - padded).astype(jnp.int32)
    te = jnp.searchsorted(ends, jnp.arange(NT, dtype=jnp.int32) * BM,
                          side="right").astype(jnp.int32)
    te = jnp.minimum(te, N_EXP - 1)

    gid, dest = _binning(routes, base)
    xsw = _gather(xw, gid)
    ysw = _ffn(xsw, We, be, te)
    outw = _scatter(ysw, dest)
    final = _finish(outw, rpm.reshape(N_TOK, 1))
    return final, counts, rps2.reshape(N_EXP), rpm


# speedup vs baseline: 2.9271x; 1.2300x over previous
import functools

import jax
import jax.numpy as jnp
from jax import lax
from jax.experimental import pallas as pl
from jax.experimental.pallas import tpu as pltpu
from jax.experimental.pallas import tpu_sc as plsc

N_TOK = 16384
N_EXP = 64
D = 768
BM = 64
MP = N_TOK + N_EXP * BM
NT = MP // BM
TRASH = N_TOK
N_PAD_ROWS = 4096
OUT_ROWS = N_TOK + N_PAD_ROWS
TB = 1024
NSUB = 16
TPW = N_TOK // NSUB
CPW = MP // NSUB
GPW = TPW // 16


_D2 = D // 2
_MASK_HI = -65536


def _pack_words(a):
    lo = lax.bitcast_convert_type(
        a[:, :_D2].astype(jnp.bfloat16).astype(jnp.float32), jnp.int32)
    hi = lax.bitcast_convert_type(
        a[:, _D2:].astype(jnp.bfloat16).astype(jnp.float32), jnp.int32)
    return lax.shift_right_logical(lo, 16) | (hi & _MASK_HI)


def _unpack_words_f32(w):
    lo = lax.bitcast_convert_type(lax.shift_left(w, 16), jnp.float32)
    hi = lax.bitcast_convert_type(w & _MASK_HI, jnp.float32)
    return lo, hi


def _router_body(x_ref, wr_ref, br_ref, routes_ref, rpm_ref, rps_ref, cnt_ref,
                 xw_ref):
    i = pl.program_id(0)
    x = x_ref[...]
    xw_ref[...] = _pack_words(x)
    wr = wr_ref[...]
    logits = lax.dot_general(x, wr, (((1,), (1,)), ((), ())),
                             preferred_element_type=jnp.float32)
    logits = logits + br_ref[...]
    prob = jax.nn.softmax(logits, axis=-1)
    rpm = jnp.max(prob, axis=-1)
    eiota = lax.broadcasted_iota(jnp.int32, (TB, N_EXP), 1)
    routes = jnp.min(jnp.where(prob == rpm[:, None], eiota, N_EXP), axis=-1)
    onehot = (eiota == routes[:, None]).astype(jnp.float32)
    routes_ref[...] = routes.reshape(TB // 128, 128)
    rpm_ref[...] = rpm.reshape(TB // 128, 128)

    @pl.when(i == 0)
    def _():
        rps_ref[...] = jnp.zeros_like(rps_ref)
        cnt_ref[...] = jnp.zeros_like(cnt_ref)

    rps_ref[...] += jnp.sum(prob, axis=0).reshape(1, N_EXP)
    cnt_ref[...] += jnp.sum(onehot, axis=0).reshape(1, N_EXP)


def _router(x, Wr, br):
    n_blk = N_TOK // TB
    return pl.pallas_call(
        _router_body,
        grid=(n_blk,),
        in_specs=[
            pl.BlockSpec((TB, D), lambda i: (i, 0)),
            pl.BlockSpec((N_EXP, D), lambda i: (0, 0)),
            pl.BlockSpec((1, N_EXP), lambda i: (0, 0)),
        ],
        out_specs=[
            pl.BlockSpec((TB // 128, 128), lambda i: (i, 0)),
            pl.BlockSpec((TB // 128, 128), lambda i: (i, 0)),
            pl.BlockSpec((1, N_EXP), lambda i: (0, 0)),
            pl.BlockSpec((1, N_EXP), lambda i: (0, 0)),
            pl.BlockSpec((TB, _D2), lambda i: (i, 0)),
        ],
        out_shape=[
            jax.ShapeDtypeStruct((N_TOK // 128, 128), jnp.int32),
            jax.ShapeDtypeStruct((N_TOK // 128, 128), jnp.float32),
            jax.ShapeDtypeStruct((1, N_EXP), jnp.float32),
            jax.ShapeDtypeStruct((1, N_EXP), jnp.float32),
            jax.ShapeDtypeStruct((N_TOK, _D2), jnp.int32),
        ],
    )(x, Wr, br.reshape(1, N_EXP))


def _bin_kernel_body(routes_hbm, base_hbm,
                     gid_hbm, dest_hbm,
                     routes_v, hist_v, cnt_v, allh_v,
                     base_v, zero_v, trash_v, slots_v, gvals_v,
                     hist_s, cnt_s, hist_sh, sem):
    s = lax.axis_index("s")
    tok0 = s * TPW
    cb = s * CPW
    iot = lax.iota(jnp.int32, 16)

    @pl.loop(0, CPW // 16)
    def _(k):
        v = iot + (cb + 16 * k)
        zero_v[pl.ds(16 * k, 16)] = v & (N_TOK - 1)
        trash_v[pl.ds(16 * k, 16)] = TRASH + (v & (N_PAD_ROWS - 1))

    init0 = pltpu.async_copy(zero_v, gid_hbm.at[pl.ds(cb, CPW)], sem)
    init1 = pltpu.async_copy(trash_v, dest_hbm.at[pl.ds(cb, CPW)], sem)

    pltpu.sync_copy(routes_hbm.at[pl.ds(tok0, TPW)], routes_v)
    pltpu.sync_copy(base_hbm, base_v)

    for l in range(N_EXP):
        hist_s[l] = 0

    @pl.loop(0, TPW // 16)
    def _(g):
        ev = routes_v[pl.ds(g * 16, 16)]
        for l in range(16):
            e = ev[l]
            hist_s[e] = hist_s[e] + 1

    for j in range(4):
        acc = jnp.zeros((16,), jnp.int32)
        for l in range(16):
            hs = hist_s[j * 16 + l]
            acc = jnp.where(iot == l, lax.broadcast(hs, (16,)), acc)
        hist_v[pl.ds(j * 16, 16)] = acc

    pltpu.sync_copy(hist_v, hist_sh.at[pl.ds(s * N_EXP, N_EXP)])
    init0.wait()
    init1.wait()
    plsc.subcore_barrier()
    pltpu.sync_copy(hist_sh, allh_v)

    for j in range(4):
        pre = jnp.zeros((16,), jnp.int32)
        for sp in range(NSUB - 1):
            h = allh_v[pl.ds(sp * N_EXP + j * 16, 16)]
            m = lax.broadcast((s > sp).astype(jnp.int32), (16,))
            pre = pre + h * m
        cv = base_v[pl.ds(j * 16, 16)] + pre
        cnt_v[pl.ds(j * 16, 16)] = cv
        for l in range(16):
            cnt_s[j * 16 + l] = cv[l]

    for r in range(8):
        @pl.loop(0, 8)
        def _(g2, r=r):
            g = r * 8 + g2
            ev = routes_v[pl.ds(g * 16, 16)]
            slot_acc = jnp.zeros((16,), jnp.int32)
            for l in range(16):
                e = ev[l]
                sl = cnt_s[e]
                cnt_s[e] = sl + 1
                slot_acc = jnp.where(iot == l, lax.broadcast(sl, (16,)),
                                     slot_acc)
            c0 = g2 * 16
            slots_v[r, pl.ds(c0, 16)] = slot_acc
            gvals_v[r, pl.ds(c0, 16)] = iot + (tok0 + g * 16)

    for r in range(8):
        pltpu.async_copy(gvals_v.at[r], gid_hbm.at[slots_v.at[r]], sem).wait()
        pltpu.async_copy(gvals_v.at[r], dest_hbm.at[slots_v.at[r]], sem).wait()


def _binning(routes, base):
    mesh = plsc.VectorSubcoreMesh(core_axis_name="c", subcore_axis_name="s",
                                  num_cores=1)
    f = pl.kernel(
        _bin_kernel_body,
        out_type=[
            jax.ShapeDtypeStruct((MP,), jnp.int32),
            jax.ShapeDtypeStruct((MP,), jnp.int32),
        ],
        mesh=mesh,
        scratch_types=[
            pltpu.VMEM((TPW,), jnp.int32),
            pltpu.VMEM((N_EXP,), jnp.int32),
            pltpu.VMEM((N_EXP,), jnp.int32),
            pltpu.VMEM((NSUB * N_EXP,), jnp.int32),
            pltpu.VMEM((N_EXP,), jnp.int32),
            pltpu.VMEM((CPW,), jnp.int32),
            pltpu.VMEM((CPW,), jnp.int32),
            pltpu.VMEM((8, 128), jnp.int32),
            pltpu.VMEM((8, 128), jnp.int32),
            pltpu.SMEM((N_EXP,), jnp.int32),
            pltpu.SMEM((N_EXP,), jnp.int32),
            pltpu.VMEM_SHARED((NSUB * N_EXP,), jnp.int32),
            pltpu.SemaphoreType.DMA,
        ],
    )
    return f(routes, base)


_SPW = MP // 32
_CH = 32
_NCH = _SPW // _CH
_NBUF = 4


def _gather_body(x_hbm, gid2_hbm, xs_hbm, gid2_v, *bufsem):
    w = lax.axis_index("s") * 2 + lax.axis_index("c")
    pltpu.sync_copy(gid2_hbm.at[w], gid2_v)
    rows = bufsem[:_NBUF]
    gsem = bufsem[_NBUF:2 * _NBUF]
    wsem = bufsem[2 * _NBUF:3 * _NBUF]

    @pl.loop(0, _NCH, step=_NBUF)
    def _(j):
        gds = []
        for b in range(_NBUF):
            c = j + b

            @pl.when(j > 0)
            def _(b=b, c=c):
                pltpu.make_async_copy(
                    rows[b], xs_hbm.at[pl.ds(w * _SPW + c * _CH, _CH)],
                    wsem[b]).wait()

            gds.append(pltpu.async_copy(
                x_hbm.at[gid2_v.at[c]], rows[b], gsem[b]))
        for b in range(_NBUF):
            c = j + b
            gds[b].wait()
            pltpu.async_copy(
                rows[b], xs_hbm.at[pl.ds(w * _SPW + c * _CH, _CH)], wsem[b])

    for b in range(_NBUF):
        pltpu.make_async_copy(
            rows[b], xs_hbm.at[pl.ds(w * _SPW, _CH)], wsem[b]).wait()


def _gather(xw, gid):
    mesh = plsc.VectorSubcoreMesh(core_axis_name="c", subcore_axis_name="s")
    f = pl.kernel(
        _gather_body,
        out_type=jax.ShapeDtypeStruct((MP, _D2), jnp.int32),
        mesh=mesh,
        scratch_types=[
            pltpu.VMEM((_NCH, _CH), jnp.int32),
            *[pltpu.VMEM((_CH, _D2), jnp.int32) for _ in range(_NBUF)],
            *[pltpu.SemaphoreType.DMA for _ in range(2 * _NBUF)],
        ],
    )
    return f(xw, gid.reshape(32, _NCH, _CH))


def _ffn_body(te_ref, xs_ref, we_ref, be_ref, ys_ref):
    ww = xs_ref[...]
    lo, hi = _unpack_words_f32(ww)
    w = we_ref[0].astype(jnp.bfloat16)
    y = lax.dot_general(lo.astype(jnp.bfloat16), w[:, :_D2],
                        (((1,), (1,)), ((), ())),
                        preferred_element_type=jnp.float32)
    y += lax.dot_general(hi.astype(jnp.bfloat16), w[:, _D2:],
                         (((1,), (1,)), ((), ())),
                         preferred_element_type=jnp.float32)
    y = jnp.maximum(y + be_ref[0], 0.0)
    ys_ref[...] = _pack_words(y)


def _ffn(xsw, We, be, te):
    grid_spec = pltpu.PrefetchScalarGridSpec(
        num_scalar_prefetch=1,
        grid=(NT,),
        in_specs=[
            pl.BlockSpec((BM, _D2), lambda i, te: (i, 0)),
            pl.BlockSpec((1, D, D), lambda i, te: (te[i], 0, 0)),
            pl.BlockSpec((1, 1, D), lambda i, te: (te[i], 0, 0)),
        ],
        out_specs=pl.BlockSpec((BM, _D2), lambda i, te: (i, 0)),
    )
    return pl.pallas_call(
        _ffn_body,
        grid_spec=grid_spec,
        out_shape=jax.ShapeDtypeStruct((MP, _D2), jnp.int32),
    )(te, xsw, We, be.reshape(N_EXP, 1, D))


def _scatter_body(ys_hbm, did2_hbm, out_hbm, did2_v, *bufsem):
    w = lax.axis_index("s") * 2 + lax.axis_index("c")
    pltpu.sync_copy(did2_hbm.at[w], did2_v)
    rows = bufsem[:_NBUF]
    rsem = bufsem[_NBUF:2 * _NBUF]
    wsem = bufsem[2 * _NBUF:3 * _NBUF]

    @pl.loop(0, _NCH, step=_NBUF)
    def _(j):
        rds = []
        for b in range(_NBUF):
            c = j + b

            @pl.when(j > 0)
            def _(b=b, c=c):
                pltpu.make_async_copy(
                    rows[b], out_hbm.at[did2_v.at[c]], wsem[b]).wait()

            rds.append(pltpu.async_copy(
                ys_hbm.at[pl.ds(w * _SPW + c * _CH, _CH)], rows[b], rsem[b]))
        for b in range(_NBUF):
            c = j + b
            rds[b].wait()
            pltpu.async_copy(rows[b], out_hbm.at[did2_v.at[c]], wsem[b])

    for b in range(_NBUF):
        pltpu.make_async_copy(rows[b], out_hbm.at[did2_v.at[0]],
                              wsem[b]).wait()


def _scatter(ysw, dest):
    mesh = plsc.VectorSubcoreMesh(core_axis_name="c", subcore_axis_name="s")
    f = pl.kernel(
        _scatter_body,
        out_type=jax.ShapeDtypeStruct((OUT_ROWS, _D2), jnp.int32),
        mesh=mesh,
        scratch_types=[
            pltpu.VMEM((_NCH, _CH), jnp.int32),
            *[pltpu.VMEM((_CH, _D2), jnp.int32) for _ in range(_NBUF)],
            *[pltpu.SemaphoreType.DMA for _ in range(2 * _NBUF)],
        ],
    )
    return f(ysw, dest.reshape(32, _NCH, _CH))


def _finish_body(ow_ref, rpm_ref, out_ref):
    lo, hi = _unpack_words_f32(ow_ref[...])
    scale = rpm_ref[...]
    out_ref[:, :_D2] = lo * scale
    out_ref[:, _D2:] = hi * scale


def _finish(outw, rpm_col):
    n_blk = N_TOK // TB
    return pl.pallas_call(
        _finish_body,
        grid=(n_blk,),
        in_specs=[
            pl.BlockSpec((TB, _D2), lambda i: (i, 0)),
            pl.BlockSpec((TB, 1), lambda i: (i, 0)),
        ],
        out_specs=pl.BlockSpec((TB, D), lambda i: (i, 0)),
        out_shape=jax.ShapeDtypeStruct((N_TOK, D), jnp.float32),
    )(outw, rpm_col)


def kernel(x, Wr, br, We, be):
    routes2, rpm2, rps2, cnt2, xw = _router(x, Wr, br)
    routes = routes2.reshape(N_TOK)
    rpm = rpm2.reshape(N_TOK)
    counts = cnt2.reshape(N_EXP)

    ci = counts.astype(jnp.int32)
    padded = (ci + (BM - 1)) // BM * BM
    ends = jnp.cumsum(padded)
    base = (ends - padded).astype(jnp.int32)
    te = jnp.searchsorted(ends, jnp.arange(NT, dtype=jnp.int32) * BM,
                          side="right").astype(jnp.int32)
    te = jnp.minimum(te, N_EXP - 1)

    gid, dest = _binning(routes, base)
    xsw = _gather(xw, gid)
    ysw = _ffn(xsw, We, be, te)
    outw = _scatter(ysw, dest)
    final = _finish(outw, rpm.reshape(N_TOK, 1))
    return final, counts, rps2.reshape(N_EXP), rpm


# speedup vs baseline: 4.4851x; 1.5323x over previous
import functools

import jax
import jax.numpy as jnp
from jax import lax
from jax.experimental import pallas as pl
from jax.experimental.pallas import tpu as pltpu
from jax.experimental.pallas import tpu_sc as plsc

N_TOK = 16384
N_EXP = 64
D = 768
BM = 64
MP = N_TOK + N_EXP * BM
NT = MP // BM
TRASH = N_TOK
N_PAD_ROWS = 4096
OUT_ROWS = N_TOK + N_PAD_ROWS
TB = 1024
NSUB = 16
TPW = N_TOK // NSUB
CPW = MP // NSUB
GPW = TPW // 16


_D2 = D // 2
_MASK_HI = -65536


def _pack_words(a):
    lo = lax.bitcast_convert_type(
        a[:, :_D2].astype(jnp.bfloat16).astype(jnp.float32), jnp.int32)
    hi = lax.bitcast_convert_type(
        a[:, _D2:].astype(jnp.bfloat16).astype(jnp.float32), jnp.int32)
    return lax.shift_right_logical(lo, 16) | (hi & _MASK_HI)


def _unpack_words_f32(w):
    lo = lax.bitcast_convert_type(lax.shift_left(w, 16), jnp.float32)
    hi = lax.bitcast_convert_type(w & _MASK_HI, jnp.float32)
    return lo, hi


def _router_body(x_ref, wr_ref, br_ref, routes_ref, rpm_ref, rps_ref, cnt_ref,
                 xw_ref):
    i = pl.program_id(0)
    x = x_ref[...]
    xw_ref[...] = _pack_words(x)
    wr = wr_ref[...]
    logits = lax.dot_general(x, wr, (((1,), (1,)), ((), ())),
                             preferred_element_type=jnp.float32)
    logits = logits + br_ref[...]
    prob = jax.nn.softmax(logits, axis=-1)
    rpm = jnp.max(prob, axis=-1)
    eiota = lax.broadcasted_iota(jnp.int32, (TB, N_EXP), 1)
    routes = jnp.min(jnp.where(prob == rpm[:, None], eiota, N_EXP), axis=-1)
    onehot = (eiota == routes[:, None]).astype(jnp.float32)
    routes_ref[...] = routes.reshape(TB // 128, 128)
    rpm_ref[...] = rpm.reshape(TB // 128, 128)

    @pl.when(i == 0)
    def _():
        rps_ref[...] = jnp.zeros_like(rps_ref)
        cnt_ref[...] = jnp.zeros_like(cnt_ref)

    rps_ref[...] += jnp.sum(prob, axis=0).reshape(1, N_EXP)
    cnt_ref[...] += jnp.sum(onehot, axis=0).reshape(1, N_EXP)


def _router(x, Wr, br):
    n_blk = N_TOK // TB
    return pl.pallas_call(
        _router_body,
        grid=(n_blk,),
        in_specs=[
            pl.BlockSpec((TB, D), lambda i: (i, 0)),
            pl.BlockSpec((N_EXP, D), lambda i: (0, 0)),
            pl.BlockSpec((1, N_EXP), lambda i: (0, 0)),
        ],
        out_specs=[
            pl.BlockSpec((TB // 128, 128), lambda i: (i, 0)),
            pl.BlockSpec((TB // 128, 128), lambda i: (i, 0)),
            pl.BlockSpec((1, N_EXP), lambda i: (0, 0)),
            pl.BlockSpec((1, N_EXP), lambda i: (0, 0)),
            pl.BlockSpec((TB, _D2), lambda i: (i, 0)),
        ],
        out_shape=[
            jax.ShapeDtypeStruct((N_TOK // 128, 128), jnp.int32),
            jax.ShapeDtypeStruct((N_TOK // 128, 128), jnp.float32),
            jax.ShapeDtypeStruct((1, N_EXP), jnp.float32),
            jax.ShapeDtypeStruct((1, N_EXP), jnp.float32),
            jax.ShapeDtypeStruct((N_TOK, _D2), jnp.int32),
        ],
    )(x, Wr, br.reshape(1, N_EXP))


def _bin_kernel_body(routes_hbm, base_hbm,
                     gid_hbm, dest_hbm,
                     routes_v, hist_v, cnt_v, allh_v,
                     base_v, zero_v, trash_v, slots_v, gvals_v,
                     hist_s, cnt_s, hist_sh, sem):
    s = lax.axis_index("s")
    tok0 = s * TPW
    cb = s * CPW
    iot = lax.iota(jnp.int32, 16)

    @pl.loop(0, CPW // 16)
    def _(k):
        v = iot + (cb + 16 * k)
        zero_v[pl.ds(16 * k, 16)] = v & (N_TOK - 1)
        trash_v[pl.ds(16 * k, 16)] = TRASH + (v & (N_PAD_ROWS - 1))

    init0 = pltpu.async_copy(zero_v, gid_hbm.at[pl.ds(cb, CPW)], sem)
    init1 = pltpu.async_copy(trash_v, dest_hbm.at[pl.ds(cb, CPW)], sem)

    pltpu.sync_copy(routes_hbm.at[pl.ds(tok0, TPW)], routes_v)
    pltpu.sync_copy(base_hbm, base_v)

    for l in range(N_EXP):
        hist_s[l] = 0

    @pl.loop(0, TPW // 16)
    def _(g):
        ev = routes_v[pl.ds(g * 16, 16)]
        for l in range(16):
            e = ev[l]
            hist_s[e] = hist_s[e] + 1

    for j in range(4):
        acc = jnp.zeros((16,), jnp.int32)
        for l in range(16):
            hs = hist_s[j * 16 + l]
            acc = jnp.where(iot == l, lax.broadcast(hs, (16,)), acc)
        hist_v[pl.ds(j * 16, 16)] = acc

    pltpu.sync_copy(hist_v, hist_sh.at[pl.ds(s * N_EXP, N_EXP)])
    init0.wait()
    init1.wait()
    plsc.subcore_barrier()
    pltpu.sync_copy(hist_sh, allh_v)

    for j in range(4):
        pre = jnp.zeros((16,), jnp.int32)
        for sp in range(NSUB - 1):
            h = allh_v[pl.ds(sp * N_EXP + j * 16, 16)]
            m = lax.broadcast((s > sp).astype(jnp.int32), (16,))
            pre = pre + h * m
        cv = base_v[pl.ds(j * 16, 16)] + pre
        cnt_v[pl.ds(j * 16, 16)] = cv
        for l in range(16):
            cnt_s[j * 16 + l] = cv[l]

    for r in range(8):
        @pl.loop(0, 8)
        def _(g2, r=r):
            g = r * 8 + g2
            ev = routes_v[pl.ds(g * 16, 16)]
            slot_acc = jnp.zeros((16,), jnp.int32)
            for l in range(16):
                e = ev[l]
                sl = cnt_s[e]
                cnt_s[e] = sl + 1
                slot_acc = jnp.where(iot == l, lax.broadcast(sl, (16,)),
                                     slot_acc)
            c0 = g2 * 16
            slots_v[r, pl.ds(c0, 16)] = slot_acc
            gvals_v[r, pl.ds(c0, 16)] = iot + (tok0 + g * 16)

    for r in range(8):
        pltpu.async_copy(gvals_v.at[r], gid_hbm.at[slots_v.at[r]], sem).wait()
        pltpu.async_copy(gvals_v.at[r], dest_hbm.at[slots_v.at[r]], sem).wait()


def _binning(routes, base):
    mesh = plsc.VectorSubcoreMesh(core_axis_name="c", subcore_axis_name="s",
                                  num_cores=1)
    f = pl.kernel(
        _bin_kernel_body,
        out_type=[
            jax.ShapeDtypeStruct((MP,), jnp.int32),
            jax.ShapeDtypeStruct((MP,), jnp.int32),
        ],
        mesh=mesh,
        scratch_types=[
            pltpu.VMEM((TPW,), jnp.int32),
            pltpu.VMEM((N_EXP,), jnp.int32),
            pltpu.VMEM((N_EXP,), jnp.int32),
            pltpu.VMEM((NSUB * N_EXP,), jnp.int32),
            pltpu.VMEM((N_EXP,), jnp.int32),
            pltpu.VMEM((CPW,), jnp.int32),
            pltpu.VMEM((CPW,), jnp.int32),
            pltpu.VMEM((8, 128), jnp.int32),
            pltpu.VMEM((8, 128), jnp.int32),
            pltpu.SMEM((N_EXP,), jnp.int32),
            pltpu.SMEM((N_EXP,), jnp.int32),
            pltpu.VMEM_SHARED((NSUB * N_EXP,), jnp.int32),
            pltpu.SemaphoreType.DMA,
        ],
    )
    return f(routes, base)


_SPW = MP // 32
_CH = 32
_NCH = _SPW // _CH
_NBUF = 4


def _gather_body(x_hbm, gid2_hbm, xs_hbm, gid2_v, *bufsem):
    w = lax.axis_index("s") * 2 + lax.axis_index("c")
    pltpu.sync_copy(gid2_hbm.at[w], gid2_v)
    rows = bufsem[:_NBUF]
    gsem = bufsem[_NBUF:2 * _NBUF]
    wsem = bufsem[2 * _NBUF:3 * _NBUF]

    @pl.loop(0, _NCH, step=_NBUF)
    def _(j):
        gds = []
        for b in range(_NBUF):
            c = j + b

            @pl.when(j > 0)
            def _(b=b, c=c):
                pltpu.make_async_copy(
                    rows[b], xs_hbm.at[pl.ds(w * _SPW + c * _CH, _CH)],
                    wsem[b]).wait()

            gds.append(pltpu.async_copy(
                x_hbm.at[gid2_v.at[c]], rows[b], gsem[b]))
        for b in range(_NBUF):
            c = j + b
            gds[b].wait()
            pltpu.async_copy(
                rows[b], xs_hbm.at[pl.ds(w * _SPW + c * _CH, _CH)], wsem[b])

    for b in range(_NBUF):
        pltpu.make_async_copy(
            rows[b], xs_hbm.at[pl.ds(w * _SPW, _CH)], wsem[b]).wait()


def _gather(xw, gid):
    mesh = plsc.VectorSubcoreMesh(core_axis_name="c", subcore_axis_name="s")
    f = pl.kernel(
        _gather_body,
        out_type=jax.ShapeDtypeStruct((MP, _D2), jnp.int32),
        mesh=mesh,
        scratch_types=[
            pltpu.VMEM((_NCH, _CH), jnp.int32),
            *[pltpu.VMEM((_CH, _D2), jnp.int32) for _ in range(_NBUF)],
            *[pltpu.SemaphoreType.DMA for _ in range(2 * _NBUF)],
        ],
    )
    return f(xw, gid.reshape(32, _NCH, _CH))


def _ffn_body(te_ref, xs_ref, we_ref, be_ref, ys_ref, wbf_ref):
    i = pl.program_id(0)
    changed = jnp.logical_or(i == 0, te_ref[i] != te_ref[jnp.maximum(i - 1, 0)])

    @pl.when(changed)
    def _():
        wbf_ref[...] = we_ref[0].astype(jnp.bfloat16)

    ww = xs_ref[...]
    lo, hi = _unpack_words_f32(ww)
    y = lax.dot_general(lo.astype(jnp.bfloat16), wbf_ref[:, :_D2],
                        (((1,), (1,)), ((), ())),
                        preferred_element_type=jnp.float32)
    y += lax.dot_general(hi.astype(jnp.bfloat16), wbf_ref[:, _D2:],
                         (((1,), (1,)), ((), ())),
                         preferred_element_type=jnp.float32)
    y = jnp.maximum(y + be_ref[0], 0.0)
    ys_ref[...] = _pack_words(y)


def _ffn(xsw, We, be, te):
    grid_spec = pltpu.PrefetchScalarGridSpec(
        num_scalar_prefetch=1,
        grid=(NT,),
        in_specs=[
            pl.BlockSpec((BM, _D2), lambda i, te: (i, 0)),
            pl.BlockSpec((1, D, D), lambda i, te: (te[i], 0, 0)),
            pl.BlockSpec((1, 1, D), lambda i, te: (te[i], 0, 0)),
        ],
        out_specs=pl.BlockSpec((BM, _D2), lambda i, te: (i, 0)),
        scratch_shapes=[pltpu.VMEM((D, D), jnp.bfloat16)],
    )
    return pl.pallas_call(
        _ffn_body,
        grid_spec=grid_spec,
        out_shape=jax.ShapeDtypeStruct((MP, _D2), jnp.int32),
    )(te, xsw, We, be.reshape(N_EXP, 1, D))


def _scatter_body(ys_hbm, did2_hbm, out_hbm, did2_v, *bufsem):
    w = lax.axis_index("s") * 2 + lax.axis_index("c")
    pltpu.sync_copy(did2_hbm.at[w], did2_v)
    rows = bufsem[:_NBUF]
    rsem = bufsem[_NBUF:2 * _NBUF]
    wsem = bufsem[2 * _NBUF:3 * _NBUF]

    @pl.loop(0, _NCH, step=_NBUF)
    def _(j):
        rds = []
        for b in range(_NBUF):
            c = j + b

            @pl.when(j > 0)
            def _(b=b, c=c):
                pltpu.make_async_copy(
                    rows[b], out_hbm.at[did2_v.at[c]], wsem[b]).wait()

            rds.append(pltpu.async_copy(
                ys_hbm.at[pl.ds(w * _SPW + c * _CH, _CH)], rows[b], rsem[b]))
        for b in range(_NBUF):
            c = j + b
            rds[b].wait()
            pltpu.async_copy(rows[b], out_hbm.at[did2_v.at[c]], wsem[b])

    for b in range(_NBUF):
        pltpu.make_async_copy(rows[b], out_hbm.at[did2_v.at[0]],
                              wsem[b]).wait()


def _scatter(ysw, dest):
    mesh = plsc.VectorSubcoreMesh(core_axis_name="c", subcore_axis_name="s")
    f = pl.kernel(
        _scatter_body,
        out_type=jax.ShapeDtypeStruct((OUT_ROWS, _D2), jnp.int32),
        mesh=mesh,
        scratch_types=[
            pltpu.VMEM((_NCH, _CH), jnp.int32),
            *[pltpu.VMEM((_CH, _D2), jnp.int32) for _ in range(_NBUF)],
            *[pltpu.SemaphoreType.DMA for _ in range(2 * _NBUF)],
        ],
    )
    return f(ysw, dest.reshape(32, _NCH, _CH))


def _finish_body(ow_ref, rpm_ref, out_ref):
    lo, hi = _unpack_words_f32(ow_ref[...])
    scale = rpm_ref[...]
    out_ref[:, :_D2] = lo * scale
    out_ref[:, _D2:] = hi * scale


def _finish(outw, rpm_col):
    n_blk = N_TOK // TB
    return pl.pallas_call(
        _finish_body,
        grid=(n_blk,),
        in_specs=[
            pl.BlockSpec((TB, _D2), lambda i: (i, 0)),
            pl.BlockSpec((TB, 1), lambda i: (i, 0)),
        ],
        out_specs=pl.BlockSpec((TB, D), lambda i: (i, 0)),
        out_shape=jax.ShapeDtypeStruct((N_TOK, D), jnp.float32),
    )(outw, rpm_col)


def kernel(x, Wr, br, We, be):
    routes2, rpm2, rps2, cnt2, xw = _router(x, Wr, br)
    routes = routes2.reshape(N_TOK)
    rpm = rpm2.reshape(N_TOK)
    counts = cnt2.reshape(N_EXP)

    ci = counts.astype(jnp.int32)
    padded = (ci + (BM - 1)) // BM * BM
    ends = jnp.cumsum(padded)
    base = (ends - padded).astype(jnp.int32)
    tile_start = jnp.arange(NT, dtype=jnp.int32) * BM
    te = jnp.sum((ends[None, :] <= tile_start[:, None]).astype(jnp.int32),
                 axis=1)
    te = jnp.minimum(te, N_EXP - 1)

    gid, dest = _binning(routes, base)
    xsw = _gather(xw, gid)
    ysw = _ffn(xsw, We, be, te)
    outw = _scatter(ysw, dest)
    final = _finish(outw, rpm.reshape(N_TOK, 1))
    return final, counts, rps2.reshape(N_EXP), rpm


# speedup vs baseline: 5.1775x; 1.1544x over previous
import functools

import jax
import jax.numpy as jnp
from jax import lax
from jax.experimental import pallas as pl
from jax.experimental.pallas import tpu as pltpu
from jax.experimental.pallas import tpu_sc as plsc

N_TOK = 16384
N_EXP = 64
D = 768
BM = 128
MP = N_TOK + N_EXP * BM
NT = MP // BM
TRASH = N_TOK
N_PAD_ROWS = 4096
OUT_ROWS = N_TOK + N_PAD_ROWS
TB = 1024
NSUB = 16
TPW = N_TOK // NSUB
CPW = MP // NSUB
GPW = TPW // 16


_D2 = D // 2
_MASK_HI = -65536


def _pack_words(a):
    lo = lax.bitcast_convert_type(
        a[:, :_D2].astype(jnp.bfloat16).astype(jnp.float32), jnp.int32)
    hi = lax.bitcast_convert_type(
        a[:, _D2:].astype(jnp.bfloat16).astype(jnp.float32), jnp.int32)
    return lax.shift_right_logical(lo, 16) | (hi & _MASK_HI)


def _unpack_words_f32(w):
    lo = lax.bitcast_convert_type(lax.shift_left(w, 16), jnp.float32)
    hi = lax.bitcast_convert_type(w & _MASK_HI, jnp.float32)
    return lo, hi


def _router_body(x_ref, wr_ref, br_ref, routes_ref, rpm_ref, rps_ref, cnt_ref,
                 xw_ref):
    i = pl.program_id(0)
    x = x_ref[...]
    xw_ref[...] = _pack_words(x)
    wr = wr_ref[...]
    logits = lax.dot_general(x, wr, (((1,), (1,)), ((), ())),
                             preferred_element_type=jnp.float32)
    logits = logits + br_ref[...]
    prob = jax.nn.softmax(logits, axis=-1)
    rpm = jnp.max(prob, axis=-1)
    eiota = lax.broadcasted_iota(jnp.int32, (TB, N_EXP), 1)
    routes = jnp.min(jnp.where(prob == rpm[:, None], eiota, N_EXP), axis=-1)
    onehot = (eiota == routes[:, None]).astype(jnp.float32)
    routes_ref[...] = routes.reshape(TB // 128, 128)
    rpm_ref[...] = rpm.reshape(TB // 128, 128)

    @pl.when(i == 0)
    def _():
        rps_ref[...] = jnp.zeros_like(rps_ref)
        cnt_ref[...] = jnp.zeros_like(cnt_ref)

    rps_ref[...] += jnp.sum(prob, axis=0).reshape(1, N_EXP)
    cnt_ref[...] += jnp.sum(onehot, axis=0).reshape(1, N_EXP)


def _router(x, Wr, br):
    n_blk = N_TOK // TB
    return pl.pallas_call(
        _router_body,
        grid=(n_blk,),
        in_specs=[
            pl.BlockSpec((TB, D), lambda i: (i, 0)),
            pl.BlockSpec((N_EXP, D), lambda i: (0, 0)),
            pl.BlockSpec((1, N_EXP), lambda i: (0, 0)),
        ],
        out_specs=[
            pl.BlockSpec((TB // 128, 128), lambda i: (i, 0)),
            pl.BlockSpec((TB // 128, 128), lambda i: (i, 0)),
            pl.BlockSpec((1, N_EXP), lambda i: (0, 0)),
            pl.BlockSpec((1, N_EXP), lambda i: (0, 0)),
            pl.BlockSpec((TB, _D2), lambda i: (i, 0)),
        ],
        out_shape=[
            jax.ShapeDtypeStruct((N_TOK // 128, 128), jnp.int32),
            jax.ShapeDtypeStruct((N_TOK // 128, 128), jnp.float32),
            jax.ShapeDtypeStruct((1, N_EXP), jnp.float32),
            jax.ShapeDtypeStruct((1, N_EXP), jnp.float32),
            jax.ShapeDtypeStruct((N_TOK, _D2), jnp.int32),
        ],
    )(x, Wr, br.reshape(1, N_EXP))


def _bin_kernel_body(routes_hbm, base_hbm,
                     gid_hbm, dest_hbm,
                     routes_v, hist_v, cnt_v, allh_v,
                     base_v, zero_v, trash_v, slots_v, gvals_v,
                     hist_s, cnt_s, hist_sh, sem):
    s = lax.axis_index("s")
    tok0 = s * TPW
    cb = s * CPW
    iot = lax.iota(jnp.int32, 16)

    @pl.loop(0, CPW // 16)
    def _(k):
        v = iot + (cb + 16 * k)
        zero_v[pl.ds(16 * k, 16)] = v & (N_TOK - 1)
        trash_v[pl.ds(16 * k, 16)] = TRASH + (v & (N_PAD_ROWS - 1))

    init0 = pltpu.async_copy(zero_v, gid_hbm.at[pl.ds(cb, CPW)], sem)
    init1 = pltpu.async_copy(trash_v, dest_hbm.at[pl.ds(cb, CPW)], sem)

    pltpu.sync_copy(routes_hbm.at[pl.ds(tok0, TPW)], routes_v)
    pltpu.sync_copy(base_hbm, base_v)

    for l in range(N_EXP):
        hist_s[l] = 0

    @pl.loop(0, TPW // 16)
    def _(g):
        ev = routes_v[pl.ds(g * 16, 16)]
        for l in range(16):
            e = ev[l]
            hist_s[e] = hist_s[e] + 1

    for j in range(4):
        acc = jnp.zeros((16,), jnp.int32)
        for l in range(16):
            hs = hist_s[j * 16 + l]
            acc = jnp.where(iot == l, lax.broadcast(hs, (16,)), acc)
        hist_v[pl.ds(j * 16, 16)] = acc

    pltpu.sync_copy(hist_v, hist_sh.at[pl.ds(s * N_EXP, N_EXP)])
    init0.wait()
    init1.wait()
    plsc.subcore_barrier()
    pltpu.sync_copy(hist_sh, allh_v)

    for j in range(4):
        pre = jnp.zeros((16,), jnp.int32)
        for sp in range(NSUB - 1):
            h = allh_v[pl.ds(sp * N_EXP + j * 16, 16)]
            m = lax.broadcast((s > sp).astype(jnp.int32), (16,))
            pre = pre + h * m
        cv = base_v[pl.ds(j * 16, 16)] + pre
        cnt_v[pl.ds(j * 16, 16)] = cv
        for l in range(16):
            cnt_s[j * 16 + l] = cv[l]

    for r in range(8):
        @pl.loop(0, 8)
        def _(g2, r=r):
            g = r * 8 + g2
            ev = routes_v[pl.ds(g * 16, 16)]
            slot_acc = jnp.zeros((16,), jnp.int32)
            for l in range(16):
                e = ev[l]
                sl = cnt_s[e]
                cnt_s[e] = sl + 1
                slot_acc = jnp.where(iot == l, lax.broadcast(sl, (16,)),
                                     slot_acc)
            c0 = g2 * 16
            slots_v[r, pl.ds(c0, 16)] = slot_acc
            gvals_v[r, pl.ds(c0, 16)] = iot + (tok0 + g * 16)

    for r in range(8):
        pltpu.async_copy(gvals_v.at[r], gid_hbm.at[slots_v.at[r]], sem).wait()
        pltpu.async_copy(gvals_v.at[r], dest_hbm.at[slots_v.at[r]], sem).wait()


def _binning(routes, base):
    mesh = plsc.VectorSubcoreMesh(core_axis_name="c", subcore_axis_name="s",
                                  num_cores=1)
    f = pl.kernel(
        _bin_kernel_body,
        out_type=[
            jax.ShapeDtypeStruct((MP,), jnp.int32),
            jax.ShapeDtypeStruct((MP,), jnp.int32),
        ],
        mesh=mesh,
        scratch_types=[
            pltpu.VMEM((TPW,), jnp.int32),
            pltpu.VMEM((N_EXP,), jnp.int32),
            pltpu.VMEM((N_EXP,), jnp.int32),
            pltpu.VMEM((NSUB * N_EXP,), jnp.int32),
            pltpu.VMEM((N_EXP,), jnp.int32),
            pltpu.VMEM((CPW,), jnp.int32),
            pltpu.VMEM((CPW,), jnp.int32),
            pltpu.VMEM((8, 128), jnp.int32),
            pltpu.VMEM((8, 128), jnp.int32),
            pltpu.SMEM((N_EXP,), jnp.int32),
            pltpu.SMEM((N_EXP,), jnp.int32),
            pltpu.VMEM_SHARED((NSUB * N_EXP,), jnp.int32),
            pltpu.SemaphoreType.DMA,
        ],
    )
    return f(routes, base)


_SPW = MP // 32
_CH = 32
_NCH = _SPW // _CH
_NBUF = 4


def _gather_body(x_hbm, gid2_hbm, xs_hbm, gid2_v, *bufsem):
    w = lax.axis_index("s") * 2 + lax.axis_index("c")
    pltpu.sync_copy(gid2_hbm.at[w], gid2_v)
    rows = bufsem[:_NBUF]
    gsem = bufsem[_NBUF:2 * _NBUF]
    wsem = bufsem[2 * _NBUF:3 * _NBUF]

    @pl.loop(0, _NCH, step=_NBUF)
    def _(j):
        gds = []
        for b in range(_NBUF):
            c = j + b

            @pl.when(j > 0)
            def _(b=b, c=c):
                pltpu.make_async_copy(
                    rows[b], xs_hbm.at[pl.ds(w * _SPW + c * _CH, _CH)],
                    wsem[b]).wait()

            gds.append(pltpu.async_copy(
                x_hbm.at[gid2_v.at[c]], rows[b], gsem[b]))
        for b in range(_NBUF):
            c = j + b
            gds[b].wait()
            pltpu.async_copy(
                rows[b], xs_hbm.at[pl.ds(w * _SPW + c * _CH, _CH)], wsem[b])

    for b in range(_NBUF):
        pltpu.make_async_copy(
            rows[b], xs_hbm.at[pl.ds(w * _SPW, _CH)], wsem[b]).wait()


def _gather(xw, gid):
    mesh = plsc.VectorSubcoreMesh(core_axis_name="c", subcore_axis_name="s")
    f = pl.kernel(
        _gather_body,
        out_type=jax.ShapeDtypeStruct((MP, _D2), jnp.int32),
        mesh=mesh,
        scratch_types=[
            pltpu.VMEM((_NCH, _CH), jnp.int32),
            *[pltpu.VMEM((_CH, _D2), jnp.int32) for _ in range(_NBUF)],
            *[pltpu.SemaphoreType.DMA for _ in range(2 * _NBUF)],
        ],
    )
    return f(xw, gid.reshape(32, _NCH, _CH))


def _ffn_body(te_ref, xs_ref, we_ref, be_ref, ys_ref, wbf_ref):
    i = pl.program_id(0)
    changed = jnp.logical_or(i == 0, te_ref[i] != te_ref[jnp.maximum(i - 1, 0)])

    @pl.when(changed)
    def _():
        wbf_ref[...] = we_ref[0].astype(jnp.bfloat16)

    ww = xs_ref[...]
    lo, hi = _unpack_words_f32(ww)
    xcat = jnp.concatenate([lo, hi], axis=1).astype(jnp.bfloat16)
    y = lax.dot_general(xcat, wbf_ref[...], (((1,), (1,)), ((), ())),
                        preferred_element_type=jnp.float32)
    y = jnp.maximum(y + be_ref[0], 0.0)
    ys_ref[...] = _pack_words(y)


def _ffn(xsw, We, be, te):
    grid_spec = pltpu.PrefetchScalarGridSpec(
        num_scalar_prefetch=1,
        grid=(NT,),
        in_specs=[
            pl.BlockSpec((BM, _D2), lambda i, te: (i, 0)),
            pl.BlockSpec((1, D, D), lambda i, te: (te[i], 0, 0)),
            pl.BlockSpec((1, 1, D), lambda i, te: (te[i], 0, 0)),
        ],
        out_specs=pl.BlockSpec((BM, _D2), lambda i, te: (i, 0)),
        scratch_shapes=[pltpu.VMEM((D, D), jnp.bfloat16)],
    )
    return pl.pallas_call(
        _ffn_body,
        grid_spec=grid_spec,
        out_shape=jax.ShapeDtypeStruct((MP, _D2), jnp.int32),
    )(te, xsw, We, be.reshape(N_EXP, 1, D))


def _scatter_body(ys_hbm, did2_hbm, out_hbm, did2_v, *bufsem):
    w = lax.axis_index("s") * 2 + lax.axis_index("c")
    pltpu.sync_copy(did2_hbm.at[w], did2_v)
    rows = bufsem[:_NBUF]
    rsem = bufsem[_NBUF:2 * _NBUF]
    wsem = bufsem[2 * _NBUF:3 * _NBUF]

    @pl.loop(0, _NCH, step=_NBUF)
    def _(j):
        rds = []
        for b in range(_NBUF):
            c = j + b

            @pl.when(j > 0)
            def _(b=b, c=c):
                pltpu.make_async_copy(
                    rows[b], out_hbm.at[did2_v.at[c]], wsem[b]).wait()

            rds.append(pltpu.async_copy(
                ys_hbm.at[pl.ds(w * _SPW + c * _CH, _CH)], rows[b], rsem[b]))
        for b in range(_NBUF):
            c = j + b
            rds[b].wait()
            pltpu.async_copy(rows[b], out_hbm.at[did2_v.at[c]], wsem[b])

    for b in range(_NBUF):
        pltpu.make_async_copy(rows[b], out_hbm.at[did2_v.at[0]],
                              wsem[b]).wait()


def _scatter(ysw, dest):
    mesh = plsc.VectorSubcoreMesh(core_axis_name="c", subcore_axis_name="s")
    f = pl.kernel(
        _scatter_body,
        out_type=jax.ShapeDtypeStruct((OUT_ROWS, _D2), jnp.int32),
        mesh=mesh,
        scratch_types=[
            pltpu.VMEM((_NCH, _CH), jnp.int32),
            *[pltpu.VMEM((_CH, _D2), jnp.int32) for _ in range(_NBUF)],
            *[pltpu.SemaphoreType.DMA for _ in range(2 * _NBUF)],
        ],
    )
    return f(ysw, dest.reshape(32, _NCH, _CH))


def _finish_body(ow_ref, rpm_ref, out_ref):
    lo, hi = _unpack_words_f32(ow_ref[...])
    scale = rpm_ref[...]
    out_ref[:, :_D2] = lo * scale
    out_ref[:, _D2:] = hi * scale


def _finish(outw, rpm_col):
    n_blk = N_TOK // TB
    return pl.pallas_call(
        _finish_body,
        grid=(n_blk,),
        in_specs=[
            pl.BlockSpec((TB, _D2), lambda i: (i, 0)),
            pl.BlockSpec((TB, 1), lambda i: (i, 0)),
        ],
        out_specs=pl.BlockSpec((TB, D), lambda i: (i, 0)),
        out_shape=jax.ShapeDtypeStruct((N_TOK, D), jnp.float32),
    )(outw, rpm_col)


def kernel(x, Wr, br, We, be):
    routes2, rpm2, rps2, cnt2, xw = _router(x, Wr, br)
    routes = routes2.reshape(N_TOK)
    rpm = rpm2.reshape(N_TOK)
    counts = cnt2.reshape(N_EXP)

    ci = counts.astype(jnp.int32)
    padded = (ci + (BM - 1)) // BM * BM
    ends = jnp.cumsum(padded)
    base = (ends - padded).astype(jnp.int32)
    tile_start = jnp.arange(NT, dtype=jnp.int32) * BM
    te = jnp.sum((ends[None, :] <= tile_start[:, None]).astype(jnp.int32),
                 axis=1)
    te = jnp.minimum(te, N_EXP - 1)

    gid, dest = _binning(routes, base)
    xsw = _gather(xw, gid)
    ysw = _ffn(xsw, We, be, te)
    outw = _scatter(ysw, dest)
    final = _finish(outw, rpm.reshape(N_TOK, 1))
    return final, counts, rps2.reshape(N_EXP), rpm


# speedup vs baseline: 5.7591x; 1.1123x over previous
import functools

import jax
import jax.numpy as jnp
from jax import lax
from jax.experimental import pallas as pl
from jax.experimental.pallas import tpu as pltpu
from jax.experimental.pallas import tpu_sc as plsc

N_TOK = 16384
N_EXP = 64
D = 768
BM = 256
MP = N_TOK + N_EXP * BM
NT = MP // BM
TRASH = N_TOK
N_PAD_ROWS = 4096
OUT_ROWS = N_TOK + N_PAD_ROWS
TB = 1024
NSUB = 16
TPW = N_TOK // NSUB
CPW = MP // NSUB
GPW = TPW // 16


_D2 = D // 2
_MASK_HI = -65536


def _pack_words(a):
    lo = lax.bitcast_convert_type(
        a[:, :_D2].astype(jnp.bfloat16).astype(jnp.float32), jnp.int32)
    hi = lax.bitcast_convert_type(
        a[:, _D2:].astype(jnp.bfloat16).astype(jnp.float32), jnp.int32)
    return lax.shift_right_logical(lo, 16) | (hi & _MASK_HI)


def _unpack_words_f32(w):
    lo = lax.bitcast_convert_type(lax.shift_left(w, 16), jnp.float32)
    hi = lax.bitcast_convert_type(w & _MASK_HI, jnp.float32)
    return lo, hi


def _router_body(x_ref, wr_ref, br_ref, routes_ref, rpm_ref, rps_ref, cnt_ref,
                 xw_ref):
    i = pl.program_id(0)
    x = x_ref[...]
    xw_ref[...] = _pack_words(x)
    wr = wr_ref[...]
    logits = lax.dot_general(x, wr, (((1,), (1,)), ((), ())),
                             preferred_element_type=jnp.float32)
    logits = logits + br_ref[...]
    prob = jax.nn.softmax(logits, axis=-1)
    rpm = jnp.max(prob, axis=-1)
    eiota = lax.broadcasted_iota(jnp.int32, (TB, N_EXP), 1)
    routes = jnp.min(jnp.where(prob == rpm[:, None], eiota, N_EXP), axis=-1)
    onehot = (eiota == routes[:, None]).astype(jnp.float32)
    routes_ref[...] = routes.reshape(TB // 128, 128)
    rpm_ref[...] = rpm.reshape(TB // 128, 128)

    @pl.when(i == 0)
    def _():
        rps_ref[...] = jnp.zeros_like(rps_ref)
        cnt_ref[...] = jnp.zeros_like(cnt_ref)

    rps_ref[...] += jnp.sum(prob, axis=0).reshape(1, N_EXP)
    cnt_ref[...] += jnp.sum(onehot, axis=0).reshape(1, N_EXP)


def _router(x, Wr, br):
    n_blk = N_TOK // TB
    return pl.pallas_call(
        _router_body,
        grid=(n_blk,),
        in_specs=[
            pl.BlockSpec((TB, D), lambda i: (i, 0)),
            pl.BlockSpec((N_EXP, D), lambda i: (0, 0)),
            pl.BlockSpec((1, N_EXP), lambda i: (0, 0)),
        ],
        out_specs=[
            pl.BlockSpec((TB // 128, 128), lambda i: (i, 0)),
            pl.BlockSpec((TB // 128, 128), lambda i: (i, 0)),
            pl.BlockSpec((1, N_EXP), lambda i: (0, 0)),
            pl.BlockSpec((1, N_EXP), lambda i: (0, 0)),
            pl.BlockSpec((TB, _D2), lambda i: (i, 0)),
        ],
        out_shape=[
            jax.ShapeDtypeStruct((N_TOK // 128, 128), jnp.int32),
            jax.ShapeDtypeStruct((N_TOK // 128, 128), jnp.float32),
            jax.ShapeDtypeStruct((1, N_EXP), jnp.float32),
            jax.ShapeDtypeStruct((1, N_EXP), jnp.float32),
            jax.ShapeDtypeStruct((N_TOK, _D2), jnp.int32),
        ],
    )(x, Wr, br.reshape(1, N_EXP))


def _bin_kernel_body(routes_hbm, base_hbm,
                     gid_hbm, dest_hbm,
                     routes_v, hist_v, cnt_v, allh_v,
                     base_v, zero_v, trash_v, slots_v, gvals_v,
                     hist_s, cnt_s, hist_sh, sem):
    s = lax.axis_index("s")
    tok0 = s * TPW
    cb = s * CPW
    iot = lax.iota(jnp.int32, 16)

    @pl.loop(0, CPW // 16)
    def _(k):
        v = iot + (cb + 16 * k)
        zero_v[pl.ds(16 * k, 16)] = v & (N_TOK - 1)
        trash_v[pl.ds(16 * k, 16)] = TRASH + (v & (N_PAD_ROWS - 1))

    init0 = pltpu.async_copy(zero_v, gid_hbm.at[pl.ds(cb, CPW)], sem)
    init1 = pltpu.async_copy(trash_v, dest_hbm.at[pl.ds(cb, CPW)], sem)

    pltpu.sync_copy(routes_hbm.at[pl.ds(tok0, TPW)], routes_v)
    pltpu.sync_copy(base_hbm, base_v)

    for l in range(N_EXP):
        hist_s[l] = 0

    @pl.loop(0, TPW // 16)
    def _(g):
        ev = routes_v[pl.ds(g * 16, 16)]
        for l in range(16):
            e = ev[l]
            hist_s[e] = hist_s[e] + 1

    for j in range(4):
        acc = jnp.zeros((16,), jnp.int32)
        for l in range(16):
            hs = hist_s[j * 16 + l]
            acc = jnp.where(iot == l, lax.broadcast(hs, (16,)), acc)
        hist_v[pl.ds(j * 16, 16)] = acc

    pltpu.sync_copy(hist_v, hist_sh.at[pl.ds(s * N_EXP, N_EXP)])
    init0.wait()
    init1.wait()
    plsc.subcore_barrier()
    pltpu.sync_copy(hist_sh, allh_v)

    for j in range(4):
        pre = jnp.zeros((16,), jnp.int32)
        for sp in range(NSUB - 1):
            h = allh_v[pl.ds(sp * N_EXP + j * 16, 16)]
            m = lax.broadcast((s > sp).astype(jnp.int32), (16,))
            pre = pre + h * m
        cv = base_v[pl.ds(j * 16, 16)] + pre
        cnt_v[pl.ds(j * 16, 16)] = cv
        for l in range(16):
            cnt_s[j * 16 + l] = cv[l]

    for r in range(8):
        @pl.loop(0, 8)
        def _(g2, r=r):
            g = r * 8 + g2
            ev = routes_v[pl.ds(g * 16, 16)]
            slot_acc = jnp.zeros((16,), jnp.int32)
            for l in range(16):
                e = ev[l]
                sl = cnt_s[e]
                cnt_s[e] = sl + 1
                slot_acc = jnp.where(iot == l, lax.broadcast(sl, (16,)),
                                     slot_acc)
            c0 = g2 * 16
            slots_v[r, pl.ds(c0, 16)] = slot_acc
            gvals_v[r, pl.ds(c0, 16)] = iot + (tok0 + g * 16)

    for r in range(8):
        pltpu.async_copy(gvals_v.at[r], gid_hbm.at[slots_v.at[r]], sem).wait()
        pltpu.async_copy(gvals_v.at[r], dest_hbm.at[slots_v.at[r]], sem).wait()


def _binning(routes, base):
    mesh = plsc.VectorSubcoreMesh(core_axis_name="c", subcore_axis_name="s",
                                  num_cores=1)
    f = pl.kernel(
        _bin_kernel_body,
        out_type=[
            jax.ShapeDtypeStruct((MP,), jnp.int32),
            jax.ShapeDtypeStruct((MP,), jnp.int32),
        ],
        mesh=mesh,
        scratch_types=[
            pltpu.VMEM((TPW,), jnp.int32),
            pltpu.VMEM((N_EXP,), jnp.int32),
            pltpu.VMEM((N_EXP,), jnp.int32),
            pltpu.VMEM((NSUB * N_EXP,), jnp.int32),
            pltpu.VMEM((N_EXP,), jnp.int32),
            pltpu.VMEM((CPW,), jnp.int32),
            pltpu.VMEM((CPW,), jnp.int32),
            pltpu.VMEM((8, 128), jnp.int32),
            pltpu.VMEM((8, 128), jnp.int32),
            pltpu.SMEM((N_EXP,), jnp.int32),
            pltpu.SMEM((N_EXP,), jnp.int32),
            pltpu.VMEM_SHARED((NSUB * N_EXP,), jnp.int32),
            pltpu.SemaphoreType.DMA,
        ],
    )
    return f(routes, base)


_SPW = MP // 32
_CH = 32
_NCH = _SPW // _CH
_NBUF = 4


def _gather_body(x_hbm, gid2_hbm, xs_hbm, gid2_v, *bufsem):
    w = lax.axis_index("s") * 2 + lax.axis_index("c")
    pltpu.sync_copy(gid2_hbm.at[w], gid2_v)
    rows = bufsem[:_NBUF]
    gsem = bufsem[_NBUF:2 * _NBUF]
    wsem = bufsem[2 * _NBUF:3 * _NBUF]

    @pl.loop(0, _NCH, step=_NBUF)
    def _(j):
        gds = []
        for b in range(_NBUF):
            c = j + b

            @pl.when(j > 0)
            def _(b=b, c=c):
                pltpu.make_async_copy(
                    rows[b], xs_hbm.at[pl.ds(w * _SPW + c * _CH, _CH)],
                    wsem[b]).wait()

            gds.append(pltpu.async_copy(
                x_hbm.at[gid2_v.at[c]], rows[b], gsem[b]))
        for b in range(_NBUF):
            c = j + b
            gds[b].wait()
            pltpu.async_copy(
                rows[b], xs_hbm.at[pl.ds(w * _SPW + c * _CH, _CH)], wsem[b])

    for b in range(_NBUF):
        pltpu.make_async_copy(
            rows[b], xs_hbm.at[pl.ds(w * _SPW, _CH)], wsem[b]).wait()


def _gather(xw, gid):
    mesh = plsc.VectorSubcoreMesh(core_axis_name="c", subcore_axis_name="s")
    f = pl.kernel(
        _gather_body,
        out_type=jax.ShapeDtypeStruct((MP, _D2), jnp.int32),
        mesh=mesh,
        scratch_types=[
            pltpu.VMEM((_NCH, _CH), jnp.int32),
            *[pltpu.VMEM((_CH, _D2), jnp.int32) for _ in range(_NBUF)],
            *[pltpu.SemaphoreType.DMA for _ in range(2 * _NBUF)],
        ],
    )
    return f(xw, gid.reshape(32, _NCH, _CH))


def _ffn_body(te_ref, xs_ref, we_ref, be_ref, ys_ref, wbf_ref):
    i = pl.program_id(0)
    changed = jnp.logical_or(i == 0, te_ref[i] != te_ref[jnp.maximum(i - 1, 0)])

    @pl.when(changed)
    def _():
        wbf_ref[...] = we_ref[0].astype(jnp.bfloat16)

    ww = xs_ref[...]
    lo, hi = _unpack_words_f32(ww)
    xcat = jnp.concatenate([lo, hi], axis=1).astype(jnp.bfloat16)
    y = lax.dot_general(xcat, wbf_ref[...], (((1,), (1,)), ((), ())),
                        preferred_element_type=jnp.float32)
    y = jnp.maximum(y + be_ref[0], 0.0)
    ys_ref[...] = _pack_words(y)


def _ffn(xsw, We, be, te):
    grid_spec = pltpu.PrefetchScalarGridSpec(
        num_scalar_prefetch=1,
        grid=(NT,),
        in_specs=[
            pl.BlockSpec((BM, _D2), lambda i, te: (i, 0)),
            pl.BlockSpec((1, D, D), lambda i, te: (te[i], 0, 0)),
            pl.BlockSpec((1, 1, D), lambda i, te: (te[i], 0, 0)),
        ],
        out_specs=pl.BlockSpec((BM, _D2), lambda i, te: (i, 0)),
        scratch_shapes=[pltpu.VMEM((D, D), jnp.bfloat16)],
    )
    return pl.pallas_call(
        _ffn_body,
        grid_spec=grid_spec,
        out_shape=jax.ShapeDtypeStruct((MP, _D2), jnp.int32),
    )(te, xsw, We, be.reshape(N_EXP, 1, D))


def _scatter_body(ys_hbm, did2_hbm, out_hbm, did2_v, *bufsem):
    w = lax.axis_index("s") * 2 + lax.axis_index("c")
    pltpu.sync_copy(did2_hbm.at[w], did2_v)
    rows = bufsem[:_NBUF]
    rsem = bufsem[_NBUF:2 * _NBUF]
    wsem = bufsem[2 * _NBUF:3 * _NBUF]

    @pl.loop(0, _NCH, step=_NBUF)
    def _(j):
        rds = []
        for b in range(_NBUF):
            c = j + b

            @pl.when(j > 0)
            def _(b=b, c=c):
                pltpu.make_async_copy(
                    rows[b], out_hbm.at[did2_v.at[c]], wsem[b]).wait()

            rds.append(pltpu.async_copy(
                ys_hbm.at[pl.ds(w * _SPW + c * _CH, _CH)], rows[b], rsem[b]))
        for b in range(_NBUF):
            c = j + b
            rds[b].wait()
            pltpu.async_copy(rows[b], out_hbm.at[did2_v.at[c]], wsem[b])

    for b in range(_NBUF):
        pltpu.make_async_copy(rows[b], out_hbm.at[did2_v.at[0]],
                              wsem[b]).wait()


def _scatter(ysw, dest):
    mesh = plsc.VectorSubcoreMesh(core_axis_name="c", subcore_axis_name="s")
    f = pl.kernel(
        _scatter_body,
        out_type=jax.ShapeDtypeStruct((OUT_ROWS, _D2), jnp.int32),
        mesh=mesh,
        scratch_types=[
            pltpu.VMEM((_NCH, _CH), jnp.int32),
            *[pltpu.VMEM((_CH, _D2), jnp.int32) for _ in range(_NBUF)],
            *[pltpu.SemaphoreType.DMA for _ in range(2 * _NBUF)],
        ],
    )
    return f(ysw, dest.reshape(32, _NCH, _CH))


def _finish_body(ow_ref, rpm_ref, out_ref):
    lo, hi = _unpack_words_f32(ow_ref[...])
    scale = rpm_ref[...]
    out_ref[:, :_D2] = lo * scale
    out_ref[:, _D2:] = hi * scale


def _finish(outw, rpm_col):
    n_blk = N_TOK // TB
    return pl.pallas_call(
        _finish_body,
        grid=(n_blk,),
        in_specs=[
            pl.BlockSpec((TB, _D2), lambda i: (i, 0)),
            pl.BlockSpec((TB, 1), lambda i: (i, 0)),
        ],
        out_specs=pl.BlockSpec((TB, D), lambda i: (i, 0)),
        out_shape=jax.ShapeDtypeStruct((N_TOK, D), jnp.float32),
    )(outw, rpm_col)


def kernel(x, Wr, br, We, be):
    routes2, rpm2, rps2, cnt2, xw = _router(x, Wr, br)
    routes = routes2.reshape(N_TOK)
    rpm = rpm2.reshape(N_TOK)
    counts = cnt2.reshape(N_EXP)

    ci = counts.astype(jnp.int32)
    padded = (ci + (BM - 1)) // BM * BM
    ends = jnp.cumsum(padded)
    base = (ends - padded).astype(jnp.int32)
    tile_start = jnp.arange(NT, dtype=jnp.int32) * BM
    te = jnp.sum((ends[None, :] <= tile_start[:, None]).astype(jnp.int32),
                 axis=1)
    te = jnp.minimum(te, N_EXP - 1)

    gid, dest = _binning(routes, base)
    xsw = _gather(xw, gid)
    ysw = _ffn(xsw, We, be, te)
    outw = _scatter(ysw, dest)
    final = _finish(outw, rpm.reshape(N_TOK, 1))
    return final, counts, rps2.reshape(N_EXP), rpm


# speedup vs baseline: 5.7811x; 1.0038x over previous
import functools

import jax
import jax.numpy as jnp
from jax import lax
from jax.experimental import pallas as pl
from jax.experimental.pallas import tpu as pltpu
from jax.experimental.pallas import tpu_sc as plsc

N_TOK = 16384
N_EXP = 64
D = 768
BM = 256
MP = N_TOK + N_EXP * BM
NT = MP // BM
TRASH = N_TOK
N_PAD_ROWS = 4096
OUT_ROWS = N_TOK + N_PAD_ROWS
TB = 1024
NSUB = 16
TPW = N_TOK // NSUB
CPW = MP // NSUB
GPW = TPW // 16


_D2 = D // 2
_MASK_HI = -65536


def _pack_words(a):
    lo = lax.bitcast_convert_type(
        a[:, :_D2].astype(jnp.bfloat16).astype(jnp.float32), jnp.int32)
    hi = lax.bitcast_convert_type(
        a[:, _D2:].astype(jnp.bfloat16).astype(jnp.float32), jnp.int32)
    return lax.shift_right_logical(lo, 16) | (hi & _MASK_HI)


def _unpack_words_f32(w):
    lo = lax.bitcast_convert_type(lax.shift_left(w, 16), jnp.float32)
    hi = lax.bitcast_convert_type(w & _MASK_HI, jnp.float32)
    return lo, hi


def _router_body(x_ref, wr_ref, br_ref, routes_ref, rpm_ref, rps_ref, cnt_ref,
                 xw_ref, hist_ref):
    i = pl.program_id(0)
    x = x_ref[...]
    xw_ref[...] = _pack_words(x)
    wr = wr_ref[...]
    logits = lax.dot_general(x, wr, (((1,), (1,)), ((), ())),
                             preferred_element_type=jnp.float32)
    logits = logits + br_ref[...]
    prob = jax.nn.softmax(logits, axis=-1)
    rpm = jnp.max(prob, axis=-1)
    eiota = lax.broadcasted_iota(jnp.int32, (TB, N_EXP), 1)
    routes = jnp.min(jnp.where(prob == rpm[:, None], eiota, N_EXP), axis=-1)
    onehot = (eiota == routes[:, None]).astype(jnp.float32)
    routes_ref[...] = routes.reshape(TB // 128, 128)
    rpm_ref[...] = rpm.reshape(TB // 128, 128)
    blk_cnt = jnp.sum(onehot, axis=0)
    hist_ref[...] = blk_cnt.reshape(1, 1, N_EXP)

    @pl.when(i == 0)
    def _():
        rps_ref[...] = jnp.zeros_like(rps_ref)
        cnt_ref[...] = jnp.zeros_like(cnt_ref)

    rps_ref[...] += jnp.sum(prob, axis=0).reshape(1, N_EXP)
    cnt_ref[...] += blk_cnt.reshape(1, N_EXP)


def _router(x, Wr, br):
    n_blk = N_TOK // TB
    return pl.pallas_call(
        _router_body,
        grid=(n_blk,),
        in_specs=[
            pl.BlockSpec((TB, D), lambda i: (i, 0)),
            pl.BlockSpec((N_EXP, D), lambda i: (0, 0)),
            pl.BlockSpec((1, N_EXP), lambda i: (0, 0)),
        ],
        out_specs=[
            pl.BlockSpec((TB // 128, 128), lambda i: (i, 0)),
            pl.BlockSpec((TB // 128, 128), lambda i: (i, 0)),
            pl.BlockSpec((1, N_EXP), lambda i: (0, 0)),
            pl.BlockSpec((1, N_EXP), lambda i: (0, 0)),
            pl.BlockSpec((TB, _D2), lambda i: (i, 0)),
            pl.BlockSpec((1, 1, N_EXP), lambda i: (i, 0, 0)),
        ],
        out_shape=[
            jax.ShapeDtypeStruct((N_TOK // 128, 128), jnp.int32),
            jax.ShapeDtypeStruct((N_TOK // 128, 128), jnp.float32),
            jax.ShapeDtypeStruct((1, N_EXP), jnp.float32),
            jax.ShapeDtypeStruct((1, N_EXP), jnp.float32),
            jax.ShapeDtypeStruct((N_TOK, _D2), jnp.int32),
            jax.ShapeDtypeStruct((NSUB, 1, N_EXP), jnp.float32),
        ],
    )(x, Wr, br.reshape(1, N_EXP))


def _bin_kernel_body(routes_hbm, allh_hbm, base_hbm,
                     gid_hbm, dest_hbm,
                     routes_v, cnt_v, allh_v,
                     base_v, zero_v, trash_v, slots_v, gvals_v,
                     cnt_s, sem):
    s = lax.axis_index("s")
    tok0 = s * TPW
    cb = s * CPW
    iot = lax.iota(jnp.int32, 16)

    @pl.loop(0, CPW // 16)
    def _(k):
        v = iot + (cb + 16 * k)
        zero_v[pl.ds(16 * k, 16)] = v & (N_TOK - 1)
        trash_v[pl.ds(16 * k, 16)] = TRASH + (v & (N_PAD_ROWS - 1))

    init0 = pltpu.async_copy(zero_v, gid_hbm.at[pl.ds(cb, CPW)], sem)
    init1 = pltpu.async_copy(trash_v, dest_hbm.at[pl.ds(cb, CPW)], sem)

    pltpu.sync_copy(routes_hbm.at[pl.ds(tok0, TPW)], routes_v)
    pltpu.sync_copy(base_hbm, base_v)
    pltpu.sync_copy(allh_hbm, allh_v)

    for j in range(4):
        pre = jnp.zeros((16,), jnp.int32)
        for sp in range(NSUB - 1):
            h = allh_v[pl.ds(sp * N_EXP + j * 16, 16)]
            m = lax.broadcast((s > sp).astype(jnp.int32), (16,))
            pre = pre + h * m
        cv = base_v[pl.ds(j * 16, 16)] + pre
        cnt_v[pl.ds(j * 16, 16)] = cv
        for l in range(16):
            cnt_s[j * 16 + l] = cv[l]

    for r in range(8):
        @pl.loop(0, 8)
        def _(g2, r=r):
            g = r * 8 + g2
            ev = routes_v[pl.ds(g * 16, 16)]
            slot_acc = jnp.zeros((16,), jnp.int32)
            for l in range(16):
                e = ev[l]
                sl = cnt_s[e]
                cnt_s[e] = sl + 1
                slot_acc = jnp.where(iot == l, lax.broadcast(sl, (16,)),
                                     slot_acc)
            c0 = g2 * 16
            slots_v[r, pl.ds(c0, 16)] = slot_acc
            gvals_v[r, pl.ds(c0, 16)] = iot + (tok0 + g * 16)

    init0.wait()
    init1.wait()
    plsc.subcore_barrier()

    for r in range(8):
        pltpu.async_copy(gvals_v.at[r], gid_hbm.at[slots_v.at[r]], sem).wait()
        pltpu.async_copy(gvals_v.at[r], dest_hbm.at[slots_v.at[r]], sem).wait()


def _binning(routes, allh, base):
    mesh = plsc.VectorSubcoreMesh(core_axis_name="c", subcore_axis_name="s",
                                  num_cores=1)
    f = pl.kernel(
        _bin_kernel_body,
        out_type=[
            jax.ShapeDtypeStruct((MP,), jnp.int32),
            jax.ShapeDtypeStruct((MP,), jnp.int32),
        ],
        mesh=mesh,
        scratch_types=[
            pltpu.VMEM((TPW,), jnp.int32),
            pltpu.VMEM((N_EXP,), jnp.int32),
            pltpu.VMEM((NSUB * N_EXP,), jnp.int32),
            pltpu.VMEM((N_EXP,), jnp.int32),
            pltpu.VMEM((CPW,), jnp.int32),
            pltpu.VMEM((CPW,), jnp.int32),
            pltpu.VMEM((8, 128), jnp.int32),
            pltpu.VMEM((8, 128), jnp.int32),
            pltpu.SMEM((N_EXP,), jnp.int32),
            pltpu.SemaphoreType.DMA,
        ],
    )
    return f(routes, allh, base)


_SPW = MP // 32
_CH = 32
_NCH = _SPW // _CH
_NBUF = 4


def _gather_body(x_hbm, gid2_hbm, xs_hbm, gid2_v, *bufsem):
    w = lax.axis_index("s") * 2 + lax.axis_index("c")
    pltpu.sync_copy(gid2_hbm.at[w], gid2_v)
    rows = bufsem[:_NBUF]
    gsem = bufsem[_NBUF:2 * _NBUF]
    wsem = bufsem[2 * _NBUF:3 * _NBUF]

    @pl.loop(0, _NCH, step=_NBUF)
    def _(j):
        gds = []
        for b in range(_NBUF):
            c = j + b

            @pl.when(j > 0)
            def _(b=b, c=c):
                pltpu.make_async_copy(
                    rows[b], xs_hbm.at[pl.ds(w * _SPW + c * _CH, _CH)],
                    wsem[b]).wait()

            gds.append(pltpu.async_copy(
                x_hbm.at[gid2_v.at[c]], rows[b], gsem[b]))
        for b in range(_NBUF):
            c = j + b
            gds[b].wait()
            pltpu.async_copy(
                rows[b], xs_hbm.at[pl.ds(w * _SPW + c * _CH, _CH)], wsem[b])

    for b in range(_NBUF):
        pltpu.make_async_copy(
            rows[b], xs_hbm.at[pl.ds(w * _SPW, _CH)], wsem[b]).wait()


def _gather(xw, gid):
    mesh = plsc.VectorSubcoreMesh(core_axis_name="c", subcore_axis_name="s")
    f = pl.kernel(
        _gather_body,
        out_type=jax.ShapeDtypeStruct((MP, _D2), jnp.int32),
        mesh=mesh,
        scratch_types=[
            pltpu.VMEM((_NCH, _CH), jnp.int32),
            *[pltpu.VMEM((_CH, _D2), jnp.int32) for _ in range(_NBUF)],
            *[pltpu.SemaphoreType.DMA for _ in range(2 * _NBUF)],
        ],
    )
    return f(xw, gid.reshape(32, _NCH, _CH))


def _ffn_body(te_ref, xs_ref, we_ref, be_ref, ys_ref, wbf_ref):
    i = pl.program_id(0)
    changed = jnp.logical_or(i == 0, te_ref[i] != te_ref[jnp.maximum(i - 1, 0)])

    @pl.when(changed)
    def _():
        wbf_ref[...] = we_ref[0].astype(jnp.bfloat16)

    ww = xs_ref[...]
    lo, hi = _unpack_words_f32(ww)
    xcat = jnp.concatenate([lo, hi], axis=1).astype(jnp.bfloat16)
    y = lax.dot_general(xcat, wbf_ref[...], (((1,), (1,)), ((), ())),
                        preferred_element_type=jnp.float32)
    y = jnp.maximum(y + be_ref[0], 0.0)
    ys_ref[...] = _pack_words(y)


def _ffn(xsw, We, be, te):
    grid_spec = pltpu.PrefetchScalarGridSpec(
        num_scalar_prefetch=1,
        grid=(NT,),
        in_specs=[
            pl.BlockSpec((BM, _D2), lambda i, te: (i, 0)),
            pl.BlockSpec((1, D, D), lambda i, te: (te[i], 0, 0)),
            pl.BlockSpec((1, 1, D), lambda i, te: (te[i], 0, 0)),
        ],
        out_specs=pl.BlockSpec((BM, _D2), lambda i, te: (i, 0)),
        scratch_shapes=[pltpu.VMEM((D, D), jnp.bfloat16)],
    )
    return pl.pallas_call(
        _ffn_body,
        grid_spec=grid_spec,
        out_shape=jax.ShapeDtypeStruct((MP, _D2), jnp.int32),
    )(te, xsw, We, be.reshape(N_EXP, 1, D))


def _scatter_body(ys_hbm, did2_hbm, out_hbm, did2_v, *bufsem):
    w = lax.axis_index("s") * 2 + lax.axis_index("c")
    pltpu.sync_copy(did2_hbm.at[w], did2_v)
    rows = bufsem[:_NBUF]
    rsem = bufsem[_NBUF:2 * _NBUF]
    wsem = bufsem[2 * _NBUF:3 * _NBUF]

    @pl.loop(0, _NCH, step=_NBUF)
    def _(j):
        rds = []
        for b in range(_NBUF):
            c = j + b

            @pl.when(j > 0)
            def _(b=b, c=c):
                pltpu.make_async_copy(
                    rows[b], out_hbm.at[did2_v.at[c]], wsem[b]).wait()

            rds.append(pltpu.async_copy(
                ys_hbm.at[pl.ds(w * _SPW + c * _CH, _CH)], rows[b], rsem[b]))
        for b in range(_NBUF):
            c = j + b
            rds[b].wait()
            pltpu.async_copy(rows[b], out_hbm.at[did2_v.at[c]], wsem[b])

    for b in range(_NBUF):
        pltpu.make_async_copy(rows[b], out_hbm.at[did2_v.at[0]],
                              wsem[b]).wait()


def _scatter(ysw, dest):
    mesh = plsc.VectorSubcoreMesh(core_axis_name="c", subcore_axis_name="s")
    f = pl.kernel(
        _scatter_body,
        out_type=jax.ShapeDtypeStruct((OUT_ROWS, _D2), jnp.int32),
        mesh=mesh,
        scratch_types=[
            pltpu.VMEM((_NCH, _CH), jnp.int32),
            *[pltpu.VMEM((_CH, _D2), jnp.int32) for _ in range(_NBUF)],
            *[pltpu.SemaphoreType.DMA for _ in range(2 * _NBUF)],
        ],
    )
    return f(ysw, dest.reshape(32, _NCH, _CH))


def _finish_body(ow_ref, rpm_ref, out_ref):
    lo, hi = _unpack_words_f32(ow_ref[...])
    scale = rpm_ref[...]
    out_ref[:, :_D2] = lo * scale
    out_ref[:, _D2:] = hi * scale


def _finish(outw, rpm_col):
    n_blk = N_TOK // TB
    return pl.pallas_call(
        _finish_body,
        grid=(n_blk,),
        in_specs=[
            pl.BlockSpec((TB, _D2), lambda i: (i, 0)),
            pl.BlockSpec((TB, 1), lambda i: (i, 0)),
        ],
        out_specs=pl.BlockSpec((TB, D), lambda i: (i, 0)),
        out_shape=jax.ShapeDtypeStruct((N_TOK, D), jnp.float32),
    )(outw, rpm_col)


def kernel(x, Wr, br, We, be):
    routes2, rpm2, rps2, cnt2, xw, hist16 = _router(x, Wr, br)
    routes = routes2.reshape(N_TOK)
    rpm = rpm2.reshape(N_TOK)
    counts = cnt2.reshape(N_EXP)

    ci = counts.astype(jnp.int32)
    padded = (ci + (BM - 1)) // BM * BM
    ends = jnp.cumsum(padded)
    base = (ends - padded).astype(jnp.int32)
    tile_start = jnp.arange(NT, dtype=jnp.int32) * BM
    te = jnp.sum((ends[None, :] <= tile_start[:, None]).astype(jnp.int32),
                 axis=1)
    te = jnp.minimum(te, N_EXP - 1)

    allh = hist16.reshape(NSUB * N_EXP).astype(jnp.int32)
    gid, dest = _binning(routes, allh, base)
    xsw = _gather(xw, gid)
    ysw = _ffn(xsw, We, be, te)
    outw = _scatter(ysw, dest)
    final = _finish(outw, rpm.reshape(N_TOK, 1))
    return final, counts, rps2.reshape(N_EXP), rpm


# speedup vs baseline: 6.8043x; 1.1770x over previous
import functools

import jax
import jax.numpy as jnp
from jax import lax
from jax.experimental import pallas as pl
from jax.experimental.pallas import tpu as pltpu
from jax.experimental.pallas import tpu_sc as plsc

N_TOK = 16384
N_EXP = 64
D = 768
BM = 256
MP = N_TOK + N_EXP * BM
NT = MP // BM
TRASH = N_TOK
N_PAD_ROWS = 4096
OUT_ROWS = N_TOK + N_PAD_ROWS
TB = 1024
NSUB = 16
TPW = N_TOK // NSUB
CPW = MP // NSUB
GPW = TPW // 16


_D2 = D // 2
_MASK_HI = -65536


def _pack_words(a):
    lo = lax.bitcast_convert_type(
        a[:, :_D2].astype(jnp.bfloat16).astype(jnp.float32), jnp.int32)
    hi = lax.bitcast_convert_type(
        a[:, _D2:].astype(jnp.bfloat16).astype(jnp.float32), jnp.int32)
    return lax.shift_right_logical(lo, 16) | (hi & _MASK_HI)


def _unpack_words_f32(w):
    lo = lax.bitcast_convert_type(lax.shift_left(w, 16), jnp.float32)
    hi = lax.bitcast_convert_type(w & _MASK_HI, jnp.float32)
    return lo, hi


def _router_body(x_ref, wr_ref, br_ref, routes_ref, rpm_ref, rps_ref, cnt_ref,
                 xw_ref, hist_ref):
    i = pl.program_id(0)
    x = x_ref[...]
    xw_ref[...] = _pack_words(x)
    wr = wr_ref[...]
    logits = lax.dot_general(x, wr, (((1,), (1,)), ((), ())),
                             preferred_element_type=jnp.float32)
    logits = logits + br_ref[...]
    prob = jax.nn.softmax(logits, axis=-1)
    rpm = jnp.max(prob, axis=-1)
    eiota = lax.broadcasted_iota(jnp.int32, (TB, N_EXP), 1)
    routes = jnp.min(jnp.where(prob == rpm[:, None], eiota, N_EXP), axis=-1)
    onehot = (eiota == routes[:, None]).astype(jnp.float32)
    routes_ref[...] = routes.reshape(TB // 128, 128)
    rpm_ref[...] = rpm.reshape(TB // 128, 128)
    blk_cnt = jnp.sum(onehot, axis=0)
    hist_ref[...] = blk_cnt.reshape(1, 1, N_EXP)

    @pl.when(i == 0)
    def _():
        rps_ref[...] = jnp.zeros_like(rps_ref)
        cnt_ref[...] = jnp.zeros_like(cnt_ref)

    rps_ref[...] += jnp.sum(prob, axis=0).reshape(1, N_EXP)
    cnt_ref[...] += blk_cnt.reshape(1, N_EXP)


def _router(x, Wr, br):
    n_blk = N_TOK // TB
    return pl.pallas_call(
        _router_body,
        grid=(n_blk,),
        in_specs=[
            pl.BlockSpec((TB, D), lambda i: (i, 0)),
            pl.BlockSpec((N_EXP, D), lambda i: (0, 0)),
            pl.BlockSpec((1, N_EXP), lambda i: (0, 0)),
        ],
        out_specs=[
            pl.BlockSpec((TB // 128, 128), lambda i: (i, 0)),
            pl.BlockSpec((TB // 128, 128), lambda i: (i, 0)),
            pl.BlockSpec((1, N_EXP), lambda i: (0, 0)),
            pl.BlockSpec((1, N_EXP), lambda i: (0, 0)),
            pl.BlockSpec((TB, _D2), lambda i: (i, 0)),
            pl.BlockSpec((1, 1, N_EXP), lambda i: (i, 0, 0)),
        ],
        out_shape=[
            jax.ShapeDtypeStruct((N_TOK // 128, 128), jnp.int32),
            jax.ShapeDtypeStruct((N_TOK // 128, 128), jnp.float32),
            jax.ShapeDtypeStruct((1, N_EXP), jnp.float32),
            jax.ShapeDtypeStruct((1, N_EXP), jnp.float32),
            jax.ShapeDtypeStruct((N_TOK, _D2), jnp.int32),
            jax.ShapeDtypeStruct((NSUB, 1, N_EXP), jnp.float32),
        ],
    )(x, Wr, br.reshape(1, N_EXP))


def _bin_kernel_body(routes_hbm, allh_hbm, base_hbm,
                     gid_hbm, dest_hbm,
                     routes_v, cnt_v, allh_v,
                     base_v, zero_v, trash_v, slots_v, gvals_v,
                     cnt_s, gid_sh, dest_sh, sem):
    s = lax.axis_index("s")
    tok0 = s * TPW
    cb = s * CPW
    iot = lax.iota(jnp.int32, 16)

    @pl.loop(0, CPW // 16)
    def _(k):
        v = iot + (cb + 16 * k)
        zero_v[pl.ds(16 * k, 16)] = v & (N_TOK - 1)
        trash_v[pl.ds(16 * k, 16)] = TRASH + (v & (N_PAD_ROWS - 1))

    init0 = pltpu.async_copy(zero_v, gid_sh.at[pl.ds(cb, CPW)], sem)
    init1 = pltpu.async_copy(trash_v, dest_sh.at[pl.ds(cb, CPW)], sem)

    pltpu.sync_copy(routes_hbm.at[pl.ds(tok0, TPW)], routes_v)
    pltpu.sync_copy(base_hbm, base_v)
    pltpu.sync_copy(allh_hbm, allh_v)

    for j in range(4):
        pre = jnp.zeros((16,), jnp.int32)
        for sp in range(NSUB - 1):
            h = allh_v[pl.ds(sp * N_EXP + j * 16, 16)]
            m = lax.broadcast((s > sp).astype(jnp.int32), (16,))
            pre = pre + h * m
        cv = base_v[pl.ds(j * 16, 16)] + pre
        cnt_v[pl.ds(j * 16, 16)] = cv
        for l in range(16):
            cnt_s[j * 16 + l] = cv[l]

    for r in range(8):
        @pl.loop(0, 8)
        def _(g2, r=r):
            g = r * 8 + g2
            ev = routes_v[pl.ds(g * 16, 16)]
            slot_acc = jnp.zeros((16,), jnp.int32)
            for l in range(16):
                e = ev[l]
                sl = cnt_s[e]
                cnt_s[e] = sl + 1
                slot_acc = jnp.where(iot == l, lax.broadcast(sl, (16,)),
                                     slot_acc)
            c0 = g2 * 16
            slots_v[r, pl.ds(c0, 16)] = slot_acc
            gvals_v[r, pl.ds(c0, 16)] = iot + (tok0 + g * 16)

    init0.wait()
    init1.wait()
    plsc.subcore_barrier()

    for r in range(8):
        pltpu.sync_copy(gvals_v.at[r], gid_sh.at[slots_v.at[r]])
        pltpu.sync_copy(gvals_v.at[r], dest_sh.at[slots_v.at[r]])
    plsc.subcore_barrier()

    pltpu.sync_copy(gid_sh.at[pl.ds(cb, CPW)], gid_hbm.at[pl.ds(cb, CPW)])
    pltpu.sync_copy(dest_sh.at[pl.ds(cb, CPW)], dest_hbm.at[pl.ds(cb, CPW)])


def _binning(routes, allh, base):
    mesh = plsc.VectorSubcoreMesh(core_axis_name="c", subcore_axis_name="s",
                                  num_cores=1)
    f = pl.kernel(
        _bin_kernel_body,
        out_type=[
            jax.ShapeDtypeStruct((MP,), jnp.int32),
            jax.ShapeDtypeStruct((MP,), jnp.int32),
        ],
        mesh=mesh,
        scratch_types=[
            pltpu.VMEM((TPW,), jnp.int32),
            pltpu.VMEM((N_EXP,), jnp.int32),
            pltpu.VMEM((NSUB * N_EXP,), jnp.int32),
            pltpu.VMEM((N_EXP,), jnp.int32),
            pltpu.VMEM((CPW,), jnp.int32),
            pltpu.VMEM((CPW,), jnp.int32),
            pltpu.VMEM((8, 128), jnp.int32),
            pltpu.VMEM((8, 128), jnp.int32),
            pltpu.SMEM((N_EXP,), jnp.int32),
            pltpu.VMEM_SHARED((MP,), jnp.int32),
            pltpu.VMEM_SHARED((MP,), jnp.int32),
            pltpu.SemaphoreType.DMA,
        ],
    )
    return f(routes, allh, base)


_SPW = MP // 32
_CH = 32
_NCH = _SPW // _CH
_NBUF = 4


def _gather_body(x_hbm, gid2_hbm, xs_hbm, gid2_v, *bufsem):
    w = lax.axis_index("s") * 2 + lax.axis_index("c")
    pltpu.sync_copy(gid2_hbm.at[w], gid2_v)
    rows = bufsem[:_NBUF]
    gsem = bufsem[_NBUF:2 * _NBUF]
    wsem = bufsem[2 * _NBUF:3 * _NBUF]

    @pl.loop(0, _NCH, step=_NBUF)
    def _(j):
        gds = []
        for b in range(_NBUF):
            c = j + b

            @pl.when(j > 0)
            def _(b=b, c=c):
                pltpu.make_async_copy(
                    rows[b], xs_hbm.at[pl.ds(w * _SPW + c * _CH, _CH)],
                    wsem[b]).wait()

            gds.append(pltpu.async_copy(
                x_hbm.at[gid2_v.at[c]], rows[b], gsem[b]))
        for b in range(_NBUF):
            c = j + b
            gds[b].wait()
            pltpu.async_copy(
                rows[b], xs_hbm.at[pl.ds(w * _SPW + c * _CH, _CH)], wsem[b])

    for b in range(_NBUF):
        pltpu.make_async_copy(
            rows[b], xs_hbm.at[pl.ds(w * _SPW, _CH)], wsem[b]).wait()


def _gather(xw, gid):
    mesh = plsc.VectorSubcoreMesh(core_axis_name="c", subcore_axis_name="s")
    f = pl.kernel(
        _gather_body,
        out_type=jax.ShapeDtypeStruct((MP, _D2), jnp.int32),
        mesh=mesh,
        scratch_types=[
            pltpu.VMEM((_NCH, _CH), jnp.int32),
            *[pltpu.VMEM((_CH, _D2), jnp.int32) for _ in range(_NBUF)],
            *[pltpu.SemaphoreType.DMA for _ in range(2 * _NBUF)],
        ],
    )
    return f(xw, gid.reshape(32, _NCH, _CH))


def _ffn_body(te_ref, xs_ref, we_ref, be_ref, ys_ref, wbf_ref):
    i = pl.program_id(0)
    changed = jnp.logical_or(i == 0, te_ref[i] != te_ref[jnp.maximum(i - 1, 0)])

    @pl.when(changed)
    def _():
        wbf_ref[...] = we_ref[0].astype(jnp.bfloat16)

    ww = xs_ref[...]
    lo, hi = _unpack_words_f32(ww)
    xcat = jnp.concatenate([lo, hi], axis=1).astype(jnp.bfloat16)
    y = lax.dot_general(xcat, wbf_ref[...], (((1,), (1,)), ((), ())),
                        preferred_element_type=jnp.float32)
    y = jnp.maximum(y + be_ref[0], 0.0)
    ys_ref[...] = _pack_words(y)


def _ffn(xsw, We, be, te):
    grid_spec = pltpu.PrefetchScalarGridSpec(
        num_scalar_prefetch=1,
        grid=(NT,),
        in_specs=[
            pl.BlockSpec((BM, _D2), lambda i, te: (i, 0)),
            pl.BlockSpec((1, D, D), lambda i, te: (te[i], 0, 0)),
            pl.BlockSpec((1, 1, D), lambda i, te: (te[i], 0, 0)),
        ],
        out_specs=pl.BlockSpec((BM, _D2), lambda i, te: (i, 0)),
        scratch_shapes=[pltpu.VMEM((D, D), jnp.bfloat16)],
    )
    return pl.pallas_call(
        _ffn_body,
        grid_spec=grid_spec,
        out_shape=jax.ShapeDtypeStruct((MP, _D2), jnp.int32),
    )(te, xsw, We, be.reshape(N_EXP, 1, D))


def _scatter_body(ys_hbm, did2_hbm, out_hbm, did2_v, *bufsem):
    w = lax.axis_index("s") * 2 + lax.axis_index("c")
    pltpu.sync_copy(did2_hbm.at[w], did2_v)
    rows = bufsem[:_NBUF]
    rsem = bufsem[_NBUF:2 * _NBUF]
    wsem = bufsem[2 * _NBUF:3 * _NBUF]

    @pl.loop(0, _NCH, step=_NBUF)
    def _(j):
        rds = []
        for b in range(_NBUF):
            c = j + b

            @pl.when(j > 0)
            def _(b=b, c=c):
                pltpu.make_async_copy(
                    rows[b], out_hbm.at[did2_v.at[c]], wsem[b]).wait()

            rds.append(pltpu.async_copy(
                ys_hbm.at[pl.ds(w * _SPW + c * _CH, _CH)], rows[b], rsem[b]))
        for b in range(_NBUF):
            c = j + b
            rds[b].wait()
            pltpu.async_copy(rows[b], out_hbm.at[did2_v.at[c]], wsem[b])

    for b in range(_NBUF):
        pltpu.make_async_copy(rows[b], out_hbm.at[did2_v.at[0]],
                              wsem[b]).wait()


def _scatter(ysw, dest):
    mesh = plsc.VectorSubcoreMesh(core_axis_name="c", subcore_axis_name="s")
    f = pl.kernel(
        _scatter_body,
        out_type=jax.ShapeDtypeStruct((OUT_ROWS, _D2), jnp.int32),
        mesh=mesh,
        scratch_types=[
            pltpu.VMEM((_NCH, _CH), jnp.int32),
            *[pltpu.VMEM((_CH, _D2), jnp.int32) for _ in range(_NBUF)],
            *[pltpu.SemaphoreType.DMA for _ in range(2 * _NBUF)],
        ],
    )
    return f(ysw, dest.reshape(32, _NCH, _CH))


def _finish_body(ow_ref, rpm_ref, out_ref):
    lo, hi = _unpack_words_f32(ow_ref[...])
    scale = rpm_ref[...]
    out_ref[:, :_D2] = lo * scale
    out_ref[:, _D2:] = hi * scale


def _finish(outw, rpm_col):
    n_blk = N_TOK // TB
    return pl.pallas_call(
        _finish_body,
        grid=(n_blk,),
        in_specs=[
            pl.BlockSpec((TB, _D2), lambda i: (i, 0)),
            pl.BlockSpec((TB, 1), lambda i: (i, 0)),
        ],
        out_specs=pl.BlockSpec((TB, D), lambda i: (i, 0)),
        out_shape=jax.ShapeDtypeStruct((N_TOK, D), jnp.float32),
    )(outw, rpm_col)


def kernel(x, Wr, br, We, be):
    routes2, rpm2, rps2, cnt2, xw, hist16 = _router(x, Wr, br)
    routes = routes2.reshape(N_TOK)
    rpm = rpm2.reshape(N_TOK)
    counts = cnt2.reshape(N_EXP)

    ci = counts.astype(jnp.int32)
    padded = (ci + (BM - 1)) // BM * BM
    ends = jnp.cumsum(padded)
    base = (ends - padded).astype(jnp.int32)
    tile_start = jnp.arange(NT, dtype=jnp.int32) * BM
    te = jnp.sum((ends[None, :] <= tile_start[:, None]).astype(jnp.int32),
                 axis=1)
    te = jnp.minimum(te, N_EXP - 1)

    allh = hist16.reshape(NSUB * N_EXP).astype(jnp.int32)
    gid, dest = _binning(routes, allh, base)
    xsw = _gather(xw, gid)
    ysw = _ffn(xsw, We, be, te)
    outw = _scatter(ysw, dest)
    final = _finish(outw, rpm.reshape(N_TOK, 1))
    return final, counts, rps2.reshape(N_EXP), rpm


# speedup vs baseline: 6.9858x; 1.0267x over previous
import functools

import jax
import jax.numpy as jnp
from jax import lax
from jax.experimental import pallas as pl
from jax.experimental.pallas import tpu as pltpu
from jax.experimental.pallas import tpu_sc as plsc

N_TOK = 16384
N_EXP = 64
D = 768
BM = 256
MP = N_TOK + N_EXP * BM
NT = MP // BM
TRASH = N_TOK
N_PAD_ROWS = 4096
OUT_ROWS = N_TOK + N_PAD_ROWS
TB = 1024
NSUB = 16
TPW = N_TOK // NSUB
CPW = MP // NSUB
GPW = TPW // 16


_D2 = D // 2
_MASK_HI = -65536


def _pack_words(a):
    lo = lax.bitcast_convert_type(
        a[:, :_D2].astype(jnp.bfloat16).astype(jnp.float32), jnp.int32)
    hi = lax.bitcast_convert_type(
        a[:, _D2:].astype(jnp.bfloat16).astype(jnp.float32), jnp.int32)
    return lax.shift_right_logical(lo, 16) | (hi & _MASK_HI)


def _unpack_words_f32(w):
    lo = lax.bitcast_convert_type(lax.shift_left(w, 16), jnp.float32)
    hi = lax.bitcast_convert_type(w & _MASK_HI, jnp.float32)
    return lo, hi


def _router_body(x_ref, wr_ref, br_ref, routes_ref, rpm_ref, rps_ref, cnt_ref,
                 xw_ref, hist_ref):
    i = pl.program_id(0)
    x = x_ref[...]
    xw_ref[...] = _pack_words(x)
    wr = wr_ref[...]
    logits = lax.dot_general(x, wr, (((1,), (1,)), ((), ())),
                             preferred_element_type=jnp.float32)
    logits = logits + br_ref[...]
    prob = jax.nn.softmax(logits, axis=-1)
    rpm = jnp.max(prob, axis=-1)
    eiota = lax.broadcasted_iota(jnp.int32, (TB, N_EXP), 1)
    routes = jnp.min(jnp.where(prob == rpm[:, None], eiota, N_EXP), axis=-1)
    onehot = (eiota == routes[:, None]).astype(jnp.float32)
    routes_ref[...] = routes.reshape(TB // 128, 128)
    rpm_ref[...] = rpm.reshape(TB // 128, 128)
    blk_cnt = jnp.sum(onehot, axis=0)
    hist_ref[...] = blk_cnt.reshape(1, 1, N_EXP)

    @pl.when(i == 0)
    def _():
        rps_ref[...] = jnp.zeros_like(rps_ref)
        cnt_ref[...] = jnp.zeros_like(cnt_ref)

    rps_ref[...] += jnp.sum(prob, axis=0).reshape(1, N_EXP)
    cnt_ref[...] += blk_cnt.reshape(1, N_EXP)


def _router(x, Wr, br):
    n_blk = N_TOK // TB
    return pl.pallas_call(
        _router_body,
        grid=(n_blk,),
        in_specs=[
            pl.BlockSpec((TB, D), lambda i: (i, 0)),
            pl.BlockSpec((N_EXP, D), lambda i: (0, 0)),
            pl.BlockSpec((1, N_EXP), lambda i: (0, 0)),
        ],
        out_specs=[
            pl.BlockSpec((TB // 128, 128), lambda i: (i, 0)),
            pl.BlockSpec((TB // 128, 128), lambda i: (i, 0)),
            pl.BlockSpec((1, N_EXP), lambda i: (0, 0)),
            pl.BlockSpec((1, N_EXP), lambda i: (0, 0)),
            pl.BlockSpec((TB, _D2), lambda i: (i, 0)),
            pl.BlockSpec((1, 1, N_EXP), lambda i: (i, 0, 0)),
        ],
        out_shape=[
            jax.ShapeDtypeStruct((N_TOK // 128, 128), jnp.int32),
            jax.ShapeDtypeStruct((N_TOK // 128, 128), jnp.float32),
            jax.ShapeDtypeStruct((1, N_EXP), jnp.float32),
            jax.ShapeDtypeStruct((1, N_EXP), jnp.float32),
            jax.ShapeDtypeStruct((N_TOK, _D2), jnp.int32),
            jax.ShapeDtypeStruct((NSUB, 1, N_EXP), jnp.float32),
        ],
    )(x, Wr, br.reshape(1, N_EXP))


def _bin_kernel_body(routes_hbm, allh_hbm, base_hbm,
                     gid_hbm, dest_hbm,
                     routes_v, cnt_v, allh_v,
                     base_v, zero_v, trash_v, slots_v, gvals_v,
                     cnt_s, gid_sh, dest_sh, sem):
    s = lax.axis_index("s")
    tok0 = s * TPW
    cb = s * CPW
    iot = lax.iota(jnp.int32, 16)

    @pl.loop(0, CPW // 16)
    def _(k):
        v = iot + (cb + 16 * k)
        zero_v[pl.ds(16 * k, 16)] = v & (N_TOK - 1)
        trash_v[pl.ds(16 * k, 16)] = TRASH + (v & (N_PAD_ROWS - 1))

    init0 = pltpu.async_copy(zero_v, gid_sh.at[pl.ds(cb, CPW)], sem)
    init1 = pltpu.async_copy(trash_v, dest_sh.at[pl.ds(cb, CPW)], sem)

    pltpu.sync_copy(routes_hbm.at[pl.ds(tok0, TPW)], routes_v)
    pltpu.sync_copy(base_hbm, base_v)
    pltpu.sync_copy(allh_hbm, allh_v)

    for j in range(4):
        pre = jnp.zeros((16,), jnp.int32)
        for sp in range(NSUB - 1):
            h = allh_v[pl.ds(sp * N_EXP + j * 16, 16)]
            m = lax.broadcast((s > sp).astype(jnp.int32), (16,))
            pre = pre + h * m
        cv = base_v[pl.ds(j * 16, 16)] + pre
        cnt_v[pl.ds(j * 16, 16)] = cv
        for l in range(16):
            cnt_s[j * 16 + l] = cv[l]

    for r in range(8):
        @pl.loop(0, 8)
        def _(g2, r=r):
            g = r * 8 + g2
            ev = routes_v[pl.ds(g * 16, 16)]
            slot_acc = jnp.zeros((16,), jnp.int32)
            for l in range(16):
                e = ev[l]
                sl = cnt_s[e]
                cnt_s[e] = sl + 1
                slot_acc = jnp.where(iot == l, lax.broadcast(sl, (16,)),
                                     slot_acc)
            c0 = g2 * 16
            slots_v[r, pl.ds(c0, 16)] = slot_acc
            gvals_v[r, pl.ds(c0, 16)] = iot + (tok0 + g * 16)

    init0.wait()
    init1.wait()
    plsc.subcore_barrier()

    for r in range(8):
        pltpu.sync_copy(gvals_v.at[r], gid_sh.at[slots_v.at[r]])
        pltpu.sync_copy(gvals_v.at[r], dest_sh.at[slots_v.at[r]])
    plsc.subcore_barrier()

    pltpu.sync_copy(gid_sh.at[pl.ds(cb, CPW)], gid_hbm.at[pl.ds(cb, CPW)])
    pltpu.sync_copy(dest_sh.at[pl.ds(cb, CPW)], dest_hbm.at[pl.ds(cb, CPW)])


def _binning(routes, allh, base):
    mesh = plsc.VectorSubcoreMesh(core_axis_name="c", subcore_axis_name="s",
                                  num_cores=1)
    f = pl.kernel(
        _bin_kernel_body,
        out_type=[
            jax.ShapeDtypeStruct((MP,), jnp.int32),
            jax.ShapeDtypeStruct((MP,), jnp.int32),
        ],
        mesh=mesh,
        scratch_types=[
            pltpu.VMEM((TPW,), jnp.int32),
            pltpu.VMEM((N_EXP,), jnp.int32),
            pltpu.VMEM((NSUB * N_EXP,), jnp.int32),
            pltpu.VMEM((N_EXP,), jnp.int32),
            pltpu.VMEM((CPW,), jnp.int32),
            pltpu.VMEM((CPW,), jnp.int32),
            pltpu.VMEM((8, 128), jnp.int32),
            pltpu.VMEM((8, 128), jnp.int32),
            pltpu.SMEM((N_EXP,), jnp.int32),
            pltpu.VMEM_SHARED((MP,), jnp.int32),
            pltpu.VMEM_SHARED((MP,), jnp.int32),
            pltpu.SemaphoreType.DMA,
        ],
    )
    return f(routes, allh, base)


_SPW = MP // 32
_CH = 32
_NCH = _SPW // _CH
_NBUF = 4


def _gather_body(x_hbm, gid2_hbm, xs_hbm, gid2_v, *bufsem):
    w = lax.axis_index("s") * 2 + lax.axis_index("c")
    pltpu.sync_copy(gid2_hbm.at[w], gid2_v)
    rows = bufsem[:_NBUF]
    gsem = bufsem[_NBUF:2 * _NBUF]
    wsem = bufsem[2 * _NBUF:3 * _NBUF]

    @pl.loop(0, _NCH, step=_NBUF)
    def _(j):
        gds = []
        for b in range(_NBUF):
            c = j + b

            @pl.when(j > 0)
            def _(b=b, c=c):
                pltpu.make_async_copy(
                    rows[b], xs_hbm.at[pl.ds(w * _SPW + c * _CH, _CH)],
                    wsem[b]).wait()

            gds.append(pltpu.async_copy(
                x_hbm.at[gid2_v.at[c]], rows[b], gsem[b]))
        for b in range(_NBUF):
            c = j + b
            gds[b].wait()
            pltpu.async_copy(
                rows[b], xs_hbm.at[pl.ds(w * _SPW + c * _CH, _CH)], wsem[b])

    for b in range(_NBUF):
        pltpu.make_async_copy(
            rows[b], xs_hbm.at[pl.ds(w * _SPW, _CH)], wsem[b]).wait()


def _gather(xw, gid):
    mesh = plsc.VectorSubcoreMesh(core_axis_name="c", subcore_axis_name="s")
    f = pl.kernel(
        _gather_body,
        out_type=jax.ShapeDtypeStruct((MP, _D2), jnp.int32),
        mesh=mesh,
        scratch_types=[
            pltpu.VMEM((_NCH, _CH), jnp.int32),
            *[pltpu.VMEM((_CH, _D2), jnp.int32) for _ in range(_NBUF)],
            *[pltpu.SemaphoreType.DMA for _ in range(2 * _NBUF)],
        ],
    )
    return f(xw, gid.reshape(32, _NCH, _CH))


def _ffn_body(te_ref, xs_ref, we_ref, be_ref, ys_ref, wbf_ref):
    i = pl.program_id(0)

    @pl.when(te_ref[i] < N_EXP)
    def _():
        changed = jnp.logical_or(
            i == 0, te_ref[i] != te_ref[jnp.maximum(i - 1, 0)])

        @pl.when(changed)
        def _():
            wbf_ref[...] = we_ref[0].astype(jnp.bfloat16)

        ww = xs_ref[...]
        lo, hi = _unpack_words_f32(ww)
        xcat = jnp.concatenate([lo, hi], axis=1).astype(jnp.bfloat16)
        y = lax.dot_general(xcat, wbf_ref[...], (((1,), (1,)), ((), ())),
                            preferred_element_type=jnp.float32)
        y = jnp.maximum(y + be_ref[0], 0.0)
        ys_ref[...] = _pack_words(y)


def _ffn(xsw, We, be, te):
    grid_spec = pltpu.PrefetchScalarGridSpec(
        num_scalar_prefetch=1,
        grid=(NT,),
        in_specs=[
            pl.BlockSpec((BM, _D2), lambda i, te: (i, 0)),
            pl.BlockSpec((1, D, D),
                         lambda i, te: (jnp.minimum(te[i], N_EXP - 1), 0, 0)),
            pl.BlockSpec((1, 1, D),
                         lambda i, te: (jnp.minimum(te[i], N_EXP - 1), 0, 0)),
        ],
        out_specs=pl.BlockSpec((BM, _D2), lambda i, te: (i, 0)),
        scratch_shapes=[pltpu.VMEM((D, D), jnp.bfloat16)],
    )
    return pl.pallas_call(
        _ffn_body,
        grid_spec=grid_spec,
        out_shape=jax.ShapeDtypeStruct((MP, _D2), jnp.int32),
    )(te, xsw, We, be.reshape(N_EXP, 1, D))


def _scatter_body(ys_hbm, did2_hbm, out_hbm, did2_v, *bufsem):
    w = lax.axis_index("s") * 2 + lax.axis_index("c")
    pltpu.sync_copy(did2_hbm.at[w], did2_v)
    rows = bufsem[:_NBUF]
    rsem = bufsem[_NBUF:2 * _NBUF]
    wsem = bufsem[2 * _NBUF:3 * _NBUF]

    @pl.loop(0, _NCH, step=_NBUF)
    def _(j):
        rds = []
        for b in range(_NBUF):
            c = j + b

            @pl.when(j > 0)
            def _(b=b, c=c):
                pltpu.make_async_copy(
                    rows[b], out_hbm.at[did2_v.at[c]], wsem[b]).wait()

            rds.append(pltpu.async_copy(
                ys_hbm.at[pl.ds(w * _SPW + c * _CH, _CH)], rows[b], rsem[b]))
        for b in range(_NBUF):
            c = j + b
            rds[b].wait()
            pltpu.async_copy(rows[b], out_hbm.at[did2_v.at[c]], wsem[b])

    for b in range(_NBUF):
        pltpu.make_async_copy(rows[b], out_hbm.at[did2_v.at[0]],
                              wsem[b]).wait()


def _scatter(ysw, dest):
    mesh = plsc.VectorSubcoreMesh(core_axis_name="c", subcore_axis_name="s")
    f = pl.kernel(
        _scatter_body,
        out_type=jax.ShapeDtypeStruct((OUT_ROWS, _D2), jnp.int32),
        mesh=mesh,
        scratch_types=[
            pltpu.VMEM((_NCH, _CH), jnp.int32),
            *[pltpu.VMEM((_CH, _D2), jnp.int32) for _ in range(_NBUF)],
            *[pltpu.SemaphoreType.DMA for _ in range(2 * _NBUF)],
        ],
    )
    return f(ysw, dest.reshape(32, _NCH, _CH))


def _finish_body(ow_ref, rpm_ref, out_ref):
    lo, hi = _unpack_words_f32(ow_ref[...])
    scale = rpm_ref[...]
    out_ref[:, :_D2] = lo * scale
    out_ref[:, _D2:] = hi * scale


def _finish(outw, rpm_col):
    n_blk = N_TOK // TB
    return pl.pallas_call(
        _finish_body,
        grid=(n_blk,),
        in_specs=[
            pl.BlockSpec((TB, _D2), lambda i: (i, 0)),
            pl.BlockSpec((TB, 1), lambda i: (i, 0)),
        ],
        out_specs=pl.BlockSpec((TB, D), lambda i: (i, 0)),
        out_shape=jax.ShapeDtypeStruct((N_TOK, D), jnp.float32),
    )(outw, rpm_col)


def kernel(x, Wr, br, We, be):
    routes2, rpm2, rps2, cnt2, xw, hist16 = _router(x, Wr, br)
    routes = routes2.reshape(N_TOK)
    rpm = rpm2.reshape(N_TOK)
    counts = cnt2.reshape(N_EXP)

    ci = counts.astype(jnp.int32)
    padded = (ci + (BM - 1)) // BM * BM
    ends = jnp.cumsum(padded)
    base = (ends - padded).astype(jnp.int32)
    tile_start = jnp.arange(NT, dtype=jnp.int32) * BM
    te = jnp.sum((ends[None, :] <= tile_start[:, None]).astype(jnp.int32),
                 axis=1)

    allh = hist16.reshape(NSUB * N_EXP).astype(jnp.int32)
    gid, dest = _binning(routes, allh, base)
    xsw = _gather(xw, gid)
    ysw = _ffn(xsw, We, be, te)
    outw = _scatter(ysw, dest)
    final = _finish(outw, rpm.reshape(N_TOK, 1))
    return final, counts, rps2.reshape(N_EXP), rpm
